# Initial kernel scaffold; baseline (speedup 1.0000x reference)
import jax
import jax.numpy as jnp
from jax.experimental import pallas as pl


def kernel(x, edge_index, W1, a_src1, a_dst1, b1, W2, a_src2, a_dst2, b2):
    raise NotImplementedError("write your pallas kernel here")


# speedup vs baseline: 86.7788x; 86.7788x over previous
import functools
import jax
import jax.numpy as jnp
from jax import lax
from jax.experimental import pallas as pl
from jax.experimental.pallas import tpu as pltpu
from jax.experimental.pallas import tpu_sc as plsc

NC = 2
NS = 16
L = 16


def _tc1_body(x_ref, w1p_ref, asm_ref, adm_ref, s1_ref, d1_ref):
    h = jnp.dot(x_ref[...], w1p_ref[...], preferred_element_type=jnp.float32)
    a_s = jnp.dot(h, asm_ref[...], preferred_element_type=jnp.float32)
    a_d = jnp.dot(h, adm_ref[...], preferred_element_type=jnp.float32)
    s1_ref[...] = jnp.concatenate([h, a_s, a_d], axis=1)
    d1_ref[...] = jnp.concatenate([a_d, jnp.zeros_like(a_d)], axis=1)


def _tc1(x, w1p, asm, adm, blk, grid):
    n = x.shape[0]
    f_in = x.shape[1]
    return pl.pallas_call(
        _tc1_body,
        grid=(grid,),
        in_specs=[
            pl.BlockSpec((blk, f_in), lambda i: (i, 0)),
            pl.BlockSpec((f_in, 64), lambda i: (0, 0)),
            pl.BlockSpec((64, 8), lambda i: (0, 0)),
            pl.BlockSpec((64, 8), lambda i: (0, 0)),
        ],
        out_specs=[
            pl.BlockSpec((blk, 80), lambda i: (i, 0)),
            pl.BlockSpec((blk, 16), lambda i: (i, 0)),
        ],
        out_shape=[
            jax.ShapeDtypeStruct((n, 80), jnp.float32),
            jax.ShapeDtypeStruct((n, 16), jnp.float32),
        ],
    )(x, w1p, asm, adm)


def _tc2_body(acc_ref, w2p_ref, b1p_ref, tile_ref, as2_ref, ad2_ref, s2_ref):
    a = acc_ref[0] + acc_ref[1]
    denom = a[:, 0:8]
    num = a[:, 16:80]
    dt = jnp.dot(denom, tile_ref[...], preferred_element_type=jnp.float32)
    out1 = jnp.maximum(num / (dt + 1e-16) + b1p_ref[...], 0.0)
    h2 = jnp.dot(out1, w2p_ref[...], preferred_element_type=jnp.float32)
    a_s = h2 * as2_ref[...]
    a_d = h2 * ad2_ref[...]
    z = jnp.zeros((a.shape[0], 13), jnp.float32)
    s2_ref[...] = jnp.concatenate([h2, a_s, a_d, z], axis=1)


def _tc2(acc1, w2p, b1p, tile8, a_src2, a_dst2, blk, grid):
    n = acc1.shape[1]
    return pl.pallas_call(
        _tc2_body,
        grid=(grid,),
        in_specs=[
            pl.BlockSpec((2, blk, 80), lambda i: (0, i, 0)),
            pl.BlockSpec((64, 1), lambda i: (0, 0)),
            pl.BlockSpec((1, 64), lambda i: (0, 0)),
            pl.BlockSpec((8, 64), lambda i: (0, 0)),
            pl.BlockSpec((1, 1), lambda i: (0, 0)),
            pl.BlockSpec((1, 1), lambda i: (0, 0)),
        ],
        out_specs=pl.BlockSpec((blk, 16), lambda i: (i, 0)),
        out_shape=jax.ShapeDtypeStruct((n, 16), jnp.float32),
    )(acc1, w2p, b1p, tile8, a_src2, a_dst2)


def _tc3_body(acc_ref, b2_ref, out_ref):
    a = acc_ref[0] + acc_ref[1]
    out_ref[...] = a[:, 0:1] / (a[:, 1:2] + 1e-16) + b2_ref[...]


def _tc3(acc2, b2, blk, grid):
    n = acc2.shape[1]
    return pl.pallas_call(
        _tc3_body,
        grid=(grid,),
        in_specs=[
            pl.BlockSpec((2, blk, 16), lambda i: (0, i, 0)),
            pl.BlockSpec((1, 1), lambda i: (0, 0)),
        ],
        out_specs=pl.BlockSpec((blk, 1), lambda i: (i, 0)),
        out_shape=jax.ShapeDtypeStruct((n, 1), jnp.float32),
    )(acc2, b2)


K1 = 256
R1 = K1 // 128


def _make_sc1(n, e):
    n_chunks = e // K1
    rows_per_tile = n // NS
    mesh = plsc.VectorSubcoreMesh(core_axis_name="c", subcore_axis_name="s")

    @functools.partial(
        pl.kernel,
        out_type=jax.ShapeDtypeStruct((NC, n, 80), jnp.float32),
        mesh=mesh,
        compiler_params=pltpu.CompilerParams(use_tc_tiling_on_sc=False),
        scratch_types=[
            pltpu.VMEM((R1, 128), jnp.int32),
            pltpu.VMEM((R1, 128), jnp.int32),
            pltpu.VMEM((K1, 80), jnp.float32),
            pltpu.VMEM((K1, 16), jnp.float32),
            pltpu.VMEM((K1, 80), jnp.float32),
            pltpu.VMEM((25, 80), jnp.float32),
            pltpu.VMEM_SHARED((n, 80), jnp.float32),
            pltpu.SemaphoreType.DMA,
        ],
    )
    def sc1(s1_hbm, d1_hbm, src_hbm, dst_hbm, acc_hbm,
            src_i, dst_i, srow, drow, orow, zbuf, acc_sh, sem):
        c = lax.axis_index("c")
        s = lax.axis_index("s")
        tile = c * NS + s

        iota = lax.iota(jnp.int32, L)
        low8 = jnp.bitwise_and(iota, 7)
        zeros = jnp.zeros((L,), jnp.float32)

        def zz(i, _):
            for q in range(5):
                zbuf[i, pl.ds(q * 16, 16)] = zeros
            return 0
        lax.fori_loop(0, 25, zz, 0)

        for q in range(rows_per_tile // 25):
            pltpu.sync_copy(zbuf, acc_sh.at[pl.ds(s * rows_per_tile + q * 25, 25)])
        plsc.subcore_barrier()

        def do_chunk(cid):
            base_row = cid * R1
            pltpu.sync_copy(src_hbm.at[pl.ds(base_row, R1)], src_i)
            pltpu.sync_copy(dst_hbm.at[pl.ds(base_row, R1)], dst_i)
            cps = [pltpu.async_copy(s1_hbm.at[src_i.at[j]],
                                    srow.at[pl.ds(j * 128, 128)], sem)
                   for j in range(R1)]
            cps += [pltpu.async_copy(d1_hbm.at[dst_i.at[j]],
                                     drow.at[pl.ds(j * 128, 128)], sem)
                    for j in range(R1)]
            for cp in cps:
                cp.wait()

            def edge(e, _):
                v1 = srow[e, pl.ds(64, 16)]
                v2 = drow[e, pl.ds(0, 16)]
                al = v1 + v2
                w = jnp.exp(jnp.maximum(al, 0.2 * al))
                orow[e, pl.ds(0, 16)] = w
                wd = w.at[low8].get(mode="promise_in_bounds")
                for q in range(4):
                    orow[e, pl.ds(16 + q * 16, 16)] = (
                        srow[e, pl.ds(q * 16, 16)] * wd)
                return 0
            lax.fori_loop(0, K1, edge, 0)

            for j in range(R1):
                pltpu.sync_copy(orow.at[pl.ds(j * 128, 128)],
                                acc_sh.at[dst_i.at[j]], add=True)

        n_iter = (n_chunks + NC * NS - 1) // (NC * NS)

        def chunk_loop(k, _):
            cid = tile + k * (NC * NS)

            @pl.when(cid < n_chunks)
            def _():
                do_chunk(cid)
            return 0
        lax.fori_loop(0, n_iter, chunk_loop, 0)

        plsc.subcore_barrier()
        start = s * 624

        @pl.when(s < NS - 1)
        def _():
            pltpu.sync_copy(acc_sh.at[pl.ds(start, 624)],
                            acc_hbm.at[c, pl.ds(start, 624), :])

        @pl.when(s == NS - 1)
        def _():
            last = 624 * (NS - 1)
            pltpu.sync_copy(acc_sh.at[pl.ds(last, n - 624 * (NS - 1))],
                            acc_hbm.at[c, pl.ds(last, n - 624 * (NS - 1)), :])

    return sc1


K2 = 256
R2 = K2 // 128


def _make_sc2(n, e):
    n_chunks = e // K2
    rows_per_tile = n // NS
    mesh = plsc.VectorSubcoreMesh(core_axis_name="c", subcore_axis_name="s")

    @functools.partial(
        pl.kernel,
        out_type=jax.ShapeDtypeStruct((NC, n, 16), jnp.float32),
        mesh=mesh,
        compiler_params=pltpu.CompilerParams(use_tc_tiling_on_sc=False),
        scratch_types=[
            pltpu.VMEM((R2, 128), jnp.int32),
            pltpu.VMEM((R2, 128), jnp.int32),
            pltpu.VMEM((K2, 16), jnp.float32),
            pltpu.VMEM((K2, 16), jnp.float32),
            pltpu.VMEM((K2, 16), jnp.float32),
            pltpu.VMEM((25, 16), jnp.float32),
            pltpu.VMEM_SHARED((n, 16), jnp.float32),
            pltpu.SemaphoreType.DMA,
        ],
    )
    def sc2(s2_hbm, src_hbm, dst_hbm, acc_hbm,
            src_i, dst_i, srow, drow, orow, zbuf, acc_sh, sem):
        c = lax.axis_index("c")
        s = lax.axis_index("s")
        tile = c * NS + s

        iota = lax.iota(jnp.int32, L)
        zeros = jnp.zeros((L,), jnp.float32)
        ones_i = jnp.full((L,), 1, jnp.int32)

        def zz(i, _):
            zbuf[i, pl.ds(0, 16)] = zeros
            return 0
        lax.fori_loop(0, 25, zz, 0)

        for q in range(rows_per_tile // 25):
            pltpu.sync_copy(zbuf, acc_sh.at[pl.ds(s * rows_per_tile + q * 25, 25)])
        plsc.subcore_barrier()

        def do_chunk(cid):
            base_row = cid * R2
            pltpu.sync_copy(src_hbm.at[pl.ds(base_row, R2)], src_i)
            pltpu.sync_copy(dst_hbm.at[pl.ds(base_row, R2)], dst_i)
            cps = [pltpu.async_copy(s2_hbm.at[src_i.at[j]],
                                    srow.at[pl.ds(j * 128, 128)], sem)
                   for j in range(R2)]
            cps += [pltpu.async_copy(s2_hbm.at[dst_i.at[j]],
                                     drow.at[pl.ds(j * 128, 128)], sem)
                    for j in range(R2)]
            for cp in cps:
                cp.wait()

            def edge(e, _):
                v_s = srow[e, pl.ds(0, 16)]
                v_d = drow[e, pl.ds(0, 16)]
                b_ad = v_d.at[ones_i + 1].get(mode="promise_in_bounds")
                al = v_s + b_ad
                wv = jnp.exp(jnp.maximum(al, 0.2 * al))
                b_w = wv.at[ones_i].get(mode="promise_in_bounds")
                prod = b_w * v_s
                row = jnp.where(iota == 0, prod,
                                jnp.where(iota == 1, b_w, zeros))
                orow[e, pl.ds(0, 16)] = row
                return 0
            lax.fori_loop(0, K2, edge, 0)

            for j in range(R2):
                pltpu.sync_copy(orow.at[pl.ds(j * 128, 128)],
                                acc_sh.at[dst_i.at[j]], add=True)

        n_iter = (n_chunks + NC * NS - 1) // (NC * NS)

        def chunk_loop(k, _):
            cid = tile + k * (NC * NS)

            @pl.when(cid < n_chunks)
            def _():
                do_chunk(cid)
            return 0
        lax.fori_loop(0, n_iter, chunk_loop, 0)

        plsc.subcore_barrier()
        start = s * 624

        @pl.when(s < NS - 1)
        def _():
            pltpu.sync_copy(acc_sh.at[pl.ds(start, 624)],
                            acc_hbm.at[c, pl.ds(start, 624), :])

        @pl.when(s == NS - 1)
        def _():
            last = 624 * (NS - 1)
            pltpu.sync_copy(acc_sh.at[pl.ds(last, n - 624 * (NS - 1))],
                            acc_hbm.at[c, pl.ds(last, n - 624 * (NS - 1)), :])

    return sc2


def kernel(x, edge_index, W1, a_src1, a_dst1, b1, W2, a_src2, a_dst2, b2):
    n = x.shape[0]
    e = edge_index.shape[1]

    j = jnp.arange(64)
    perm = (j % 8) * 8 + j // 8
    w1p = W1[:, perm]
    w2p = W2[perm, :]
    b1p = b1[perm][None, :]
    eye8 = jnp.eye(8, dtype=jnp.float32)
    asm = (a_src1.T[:, :, None] * eye8[None, :, :]).reshape(64, 8)
    adm = (a_dst1.T[:, :, None] * eye8[None, :, :]).reshape(64, 8)
    tile8 = jnp.tile(eye8, (1, 8))

    src_r = edge_index[0].reshape(e // 128, 128)
    dst_r = edge_index[1].reshape(e // 128, 128)

    blk, grid = 1000, n // 1000

    s1, d1 = _tc1(x, w1p, asm, adm, blk, grid)
    acc1 = _make_sc1(n, e)(s1, d1, src_r, dst_r)
    s2 = _tc2(acc1, w2p, b1p, tile8, a_src2, a_dst2, blk, grid)
    acc2 = _make_sc2(n, e)(s2, src_r, dst_r)
    out = _tc3(acc2, b2.reshape(1, 1), blk, grid)
    return out[:, 0]


# speedup vs baseline: 136.5164x; 1.5732x over previous
import functools
import jax
import jax.numpy as jnp
from jax import lax
from jax.experimental import pallas as pl
from jax.experimental.pallas import tpu as pltpu
from jax.experimental.pallas import tpu_sc as plsc

NC = 2
NS = 16
L = 16


def _tc1_body(x_ref, w1p_ref, asm_ref, adm_ref, s1_ref, d1_ref):
    h = jnp.dot(x_ref[...], w1p_ref[...], preferred_element_type=jnp.float32)
    a_s = jnp.dot(h, asm_ref[...], preferred_element_type=jnp.float32)
    a_d = jnp.dot(h, adm_ref[...], preferred_element_type=jnp.float32)
    s1_ref[...] = jnp.concatenate([h, a_s, a_d], axis=1)
    d1_ref[...] = jnp.concatenate([a_d, jnp.zeros_like(a_d)], axis=1)


def _tc1(x, w1p, asm, adm, blk, grid):
    n = x.shape[0]
    f_in = x.shape[1]
    return pl.pallas_call(
        _tc1_body,
        grid=(grid,),
        in_specs=[
            pl.BlockSpec((blk, f_in), lambda i: (i, 0)),
            pl.BlockSpec((f_in, 64), lambda i: (0, 0)),
            pl.BlockSpec((64, 8), lambda i: (0, 0)),
            pl.BlockSpec((64, 8), lambda i: (0, 0)),
        ],
        out_specs=[
            pl.BlockSpec((blk, 80), lambda i: (i, 0)),
            pl.BlockSpec((blk, 16), lambda i: (i, 0)),
        ],
        out_shape=[
            jax.ShapeDtypeStruct((n, 80), jnp.float32),
            jax.ShapeDtypeStruct((n, 16), jnp.float32),
        ],
    )(x, w1p, asm, adm)


def _tc2_body(acc_ref, w2p_ref, b1p_ref, tile_ref, as2_ref, ad2_ref, s2_ref):
    a = acc_ref[0] + acc_ref[1]
    denom = a[:, 0:8]
    num = a[:, 16:80]
    dt = jnp.dot(denom, tile_ref[...], preferred_element_type=jnp.float32)
    out1 = jnp.maximum(num / (dt + 1e-16) + b1p_ref[...], 0.0)
    h2 = jnp.dot(out1, w2p_ref[...], preferred_element_type=jnp.float32)
    a_s = h2 * as2_ref[...]
    a_d = h2 * ad2_ref[...]
    z = jnp.zeros((a.shape[0], 13), jnp.float32)
    s2_ref[...] = jnp.concatenate([h2, a_s, a_d, z], axis=1)


def _tc2(acc1, w2p, b1p, tile8, a_src2, a_dst2, blk, grid):
    n = acc1.shape[1]
    return pl.pallas_call(
        _tc2_body,
        grid=(grid,),
        in_specs=[
            pl.BlockSpec((2, blk, 80), lambda i: (0, i, 0)),
            pl.BlockSpec((64, 1), lambda i: (0, 0)),
            pl.BlockSpec((1, 64), lambda i: (0, 0)),
            pl.BlockSpec((8, 64), lambda i: (0, 0)),
            pl.BlockSpec((1, 1), lambda i: (0, 0)),
            pl.BlockSpec((1, 1), lambda i: (0, 0)),
        ],
        out_specs=pl.BlockSpec((blk, 16), lambda i: (i, 0)),
        out_shape=jax.ShapeDtypeStruct((n, 16), jnp.float32),
    )(acc1, w2p, b1p, tile8, a_src2, a_dst2)


def _tc3_body(acc_ref, b2_ref, out_ref):
    a = acc_ref[0] + acc_ref[1]
    out_ref[...] = a[:, 0:1] / (a[:, 1:2] + 1e-16) + b2_ref[...]


def _tc3(acc2, b2, blk, grid):
    n = acc2.shape[1]
    return pl.pallas_call(
        _tc3_body,
        grid=(grid,),
        in_specs=[
            pl.BlockSpec((2, blk, 16), lambda i: (0, i, 0)),
            pl.BlockSpec((1, 1), lambda i: (0, 0)),
        ],
        out_specs=pl.BlockSpec((blk, 1), lambda i: (i, 0)),
        out_shape=jax.ShapeDtypeStruct((n, 1), jnp.float32),
    )(acc2, b2)


K1 = 128


def _make_sc1(n, e):
    n_chunks = e // K1
    rows_per_tile = n // NS
    n_iter = (n_chunks + NC * NS - 1) // (NC * NS)
    mesh = plsc.VectorSubcoreMesh(core_axis_name="c", subcore_axis_name="s")

    @functools.partial(
        pl.kernel,
        out_type=jax.ShapeDtypeStruct((NC, n, 80), jnp.float32),
        mesh=mesh,
        compiler_params=pltpu.CompilerParams(use_tc_tiling_on_sc=False),
        scratch_types=[
            pltpu.VMEM((2, 128), jnp.int32),
            pltpu.VMEM((2, 128), jnp.int32),
            pltpu.VMEM((2, K1, 80), jnp.float32),
            pltpu.VMEM((2, K1, 16), jnp.float32),
            pltpu.VMEM((K1, 80), jnp.float32),
            pltpu.VMEM((125, 80), jnp.float32),
            pltpu.VMEM_SHARED((n, 80), jnp.float32),
            pltpu.SemaphoreType.DMA((2,)),
        ],
    )
    def sc1(s1_hbm, d1_hbm, src_hbm, dst_hbm, acc_hbm,
            src_i, dst_i, srow, drow, orow, zbuf, acc_sh, sem):
        c = lax.axis_index("c")
        s = lax.axis_index("s")
        tile = c * NS + s

        iota = lax.iota(jnp.int32, L)
        low8 = jnp.bitwise_and(iota, 7)
        zeros = jnp.zeros((L,), jnp.float32)

        def zz(i, _):
            for q in range(5):
                zbuf[i, pl.ds(q * 16, 16)] = zeros
            return 0
        lax.fori_loop(0, 125, zz, 0)

        for q in range(rows_per_tile // 125):
            pltpu.sync_copy(zbuf,
                            acc_sh.at[pl.ds(s * rows_per_tile + q * 125, 125)])
        plsc.subcore_barrier()

        def fire(k, b):
            cid = tile + k * (NC * NS)

            @pl.when(cid < n_chunks)
            def _():
                pltpu.sync_copy(src_hbm.at[pl.ds(cid, 1)],
                                src_i.at[pl.ds(b, 1)])
                pltpu.sync_copy(dst_hbm.at[pl.ds(cid, 1)],
                                dst_i.at[pl.ds(b, 1)])
                pltpu.async_copy(s1_hbm.at[src_i.at[b]], srow.at[b],
                                 sem.at[b])
                pltpu.async_copy(d1_hbm.at[dst_i.at[b]], drow.at[b],
                                 sem.at[b])

        fire(0, 0)

        def chunk_loop(k, _):
            b = jnp.bitwise_and(k, 1)
            cid = tile + k * (NC * NS)
            fire(k + 1, 1 - b)

            @pl.when(cid < n_chunks)
            def _():
                pltpu.make_async_copy(s1_hbm.at[src_i.at[b]],
                                      srow.at[b], sem.at[b]).wait()
                pltpu.make_async_copy(d1_hbm.at[dst_i.at[b]],
                                      drow.at[b], sem.at[b]).wait()

                @plsc.parallel_loop(0, K1, 1, unroll=4)
                def edge(i):
                    v1 = srow[b, i, pl.ds(64, 16)]
                    v2 = drow[b, i, pl.ds(0, 16)]
                    al = v1 + v2
                    w = jnp.exp(jnp.maximum(al, 0.2 * al))
                    orow[i, pl.ds(0, 16)] = w
                    wd = w.at[low8].get(mode="promise_in_bounds")
                    for q in range(4):
                        orow[i, pl.ds(16 + q * 16, 16)] = (
                            srow[b, i, pl.ds(q * 16, 16)] * wd)

                pltpu.sync_copy(orow, acc_sh.at[dst_i.at[b]], add=True)
            return 0
        lax.fori_loop(0, n_iter, chunk_loop, 0)

        plsc.subcore_barrier()
        start = s * 624

        @pl.when(s < NS - 1)
        def _():
            pltpu.sync_copy(acc_sh.at[pl.ds(start, 624)],
                            acc_hbm.at[c, pl.ds(start, 624), :])

        @pl.when(s == NS - 1)
        def _():
            last = 624 * (NS - 1)
            pltpu.sync_copy(acc_sh.at[pl.ds(last, n - 624 * (NS - 1))],
                            acc_hbm.at[c, pl.ds(last, n - 624 * (NS - 1)), :])

    return sc1


K2 = 128


def _make_sc2(n, e):
    n_chunks = e // K2
    rows_per_tile = n // NS
    n_iter = (n_chunks + NC * NS - 1) // (NC * NS)
    mesh = plsc.VectorSubcoreMesh(core_axis_name="c", subcore_axis_name="s")

    @functools.partial(
        pl.kernel,
        out_type=jax.ShapeDtypeStruct((NC, n, 16), jnp.float32),
        mesh=mesh,
        compiler_params=pltpu.CompilerParams(use_tc_tiling_on_sc=False),
        scratch_types=[
            pltpu.VMEM((2, 128), jnp.int32),
            pltpu.VMEM((2, 128), jnp.int32),
            pltpu.VMEM((2, K2, 16), jnp.float32),
            pltpu.VMEM((2, K2, 16), jnp.float32),
            pltpu.VMEM((K2, 16), jnp.float32),
            pltpu.VMEM((125, 16), jnp.float32),
            pltpu.VMEM_SHARED((n, 16), jnp.float32),
            pltpu.SemaphoreType.DMA((2,)),
        ],
    )
    def sc2(s2_hbm, src_hbm, dst_hbm, acc_hbm,
            src_i, dst_i, srow, drow, orow, zbuf, acc_sh, sem):
        c = lax.axis_index("c")
        s = lax.axis_index("s")
        tile = c * NS + s

        iota = lax.iota(jnp.int32, L)
        zeros = jnp.zeros((L,), jnp.float32)
        ones_i = jnp.full((L,), 1, jnp.int32)

        def zz(i, _):
            zbuf[i, pl.ds(0, 16)] = zeros
            return 0
        lax.fori_loop(0, 125, zz, 0)

        for q in range(rows_per_tile // 125):
            pltpu.sync_copy(zbuf,
                            acc_sh.at[pl.ds(s * rows_per_tile + q * 125, 125)])
        plsc.subcore_barrier()

        def fire(k, b):
            cid = tile + k * (NC * NS)

            @pl.when(cid < n_chunks)
            def _():
                pltpu.sync_copy(src_hbm.at[pl.ds(cid, 1)],
                                src_i.at[pl.ds(b, 1)])
                pltpu.sync_copy(dst_hbm.at[pl.ds(cid, 1)],
                                dst_i.at[pl.ds(b, 1)])
                pltpu.async_copy(s2_hbm.at[src_i.at[b]], srow.at[b],
                                 sem.at[b])
                pltpu.async_copy(s2_hbm.at[dst_i.at[b]], drow.at[b],
                                 sem.at[b])

        fire(0, 0)

        def chunk_loop(k, _):
            b = jnp.bitwise_and(k, 1)
            cid = tile + k * (NC * NS)
            fire(k + 1, 1 - b)

            @pl.when(cid < n_chunks)
            def _():
                pltpu.make_async_copy(s2_hbm.at[src_i.at[b]],
                                      srow.at[b], sem.at[b]).wait()
                pltpu.make_async_copy(s2_hbm.at[dst_i.at[b]],
                                      drow.at[b], sem.at[b]).wait()

                @plsc.parallel_loop(0, K2, 1, unroll=8)
                def edge(i):
                    v_s = srow[b, i, pl.ds(0, 16)]
                    v_d = drow[b, i, pl.ds(0, 16)]
                    b_ad = v_d.at[ones_i + 1].get(mode="promise_in_bounds")
                    al = v_s + b_ad
                    wv = jnp.exp(jnp.maximum(al, 0.2 * al))
                    b_w = wv.at[ones_i].get(mode="promise_in_bounds")
                    prod = b_w * v_s
                    row = jnp.where(iota == 0, prod,
                                    jnp.where(iota == 1, b_w, zeros))
                    orow[i, pl.ds(0, 16)] = row

                pltpu.sync_copy(orow, acc_sh.at[dst_i.at[b]], add=True)
            return 0
        lax.fori_loop(0, n_iter, chunk_loop, 0)

        plsc.subcore_barrier()
        start = s * 624

        @pl.when(s < NS - 1)
        def _():
            pltpu.sync_copy(acc_sh.at[pl.ds(start, 624)],
                            acc_hbm.at[c, pl.ds(start, 624), :])

        @pl.when(s == NS - 1)
        def _():
            last = 624 * (NS - 1)
            pltpu.sync_copy(acc_sh.at[pl.ds(last, n - 624 * (NS - 1))],
                            acc_hbm.at[c, pl.ds(last, n - 624 * (NS - 1)), :])

    return sc2


def kernel(x, edge_index, W1, a_src1, a_dst1, b1, W2, a_src2, a_dst2, b2):
    n = x.shape[0]
    e = edge_index.shape[1]

    j = jnp.arange(64)
    perm = (j % 8) * 8 + j // 8
    w1p = W1[:, perm]
    w2p = W2[perm, :]
    b1p = b1[perm][None, :]
    eye8 = jnp.eye(8, dtype=jnp.float32)
    asm = (a_src1.T[:, :, None] * eye8[None, :, :]).reshape(64, 8)
    adm = (a_dst1.T[:, :, None] * eye8[None, :, :]).reshape(64, 8)
    tile8 = jnp.tile(eye8, (1, 8))

    src_r = edge_index[0].reshape(e // 128, 128)
    dst_r = edge_index[1].reshape(e // 128, 128)

    blk, grid = 1000, n // 1000

    s1, d1 = _tc1(x, w1p, asm, adm, blk, grid)
    acc1 = _make_sc1(n, e)(s1, d1, src_r, dst_r)
    s2 = _tc2(acc1, w2p, b1p, tile8, a_src2, a_dst2, blk, grid)
    acc2 = _make_sc2(n, e)(s2, src_r, dst_r)
    out = _tc3(acc2, b2.reshape(1, 1), blk, grid)
    return out[:, 0]


# speedup vs baseline: 197.6567x; 1.4479x over previous
import functools
import jax
import jax.numpy as jnp
from jax import lax
from jax.experimental import pallas as pl
from jax.experimental.pallas import tpu as pltpu
from jax.experimental.pallas import tpu_sc as plsc

NC = 2
NS = 16
L = 16


def _tc1_body(x_ref, w1p_ref, asm_ref, adm_ref, s1_ref, d1_ref):
    h = jnp.dot(x_ref[...], w1p_ref[...], preferred_element_type=jnp.float32)
    a_s = jnp.dot(h, asm_ref[...], preferred_element_type=jnp.float32)
    a_d = jnp.dot(h, adm_ref[...], preferred_element_type=jnp.float32)
    s1_ref[...] = jnp.concatenate([h, a_s, a_d], axis=1)
    d1_ref[...] = jnp.concatenate([a_d, jnp.zeros_like(a_d)], axis=1)


def _tc1(x, w1p, asm, adm, blk, grid):
    n = x.shape[0]
    f_in = x.shape[1]
    return pl.pallas_call(
        _tc1_body,
        grid=(grid,),
        in_specs=[
            pl.BlockSpec((blk, f_in), lambda i: (i, 0)),
            pl.BlockSpec((f_in, 64), lambda i: (0, 0)),
            pl.BlockSpec((64, 8), lambda i: (0, 0)),
            pl.BlockSpec((64, 8), lambda i: (0, 0)),
        ],
        out_specs=[
            pl.BlockSpec((blk, 80), lambda i: (i, 0)),
            pl.BlockSpec((blk, 16), lambda i: (i, 0)),
        ],
        out_shape=[
            jax.ShapeDtypeStruct((n, 80), jnp.float32),
            jax.ShapeDtypeStruct((n, 16), jnp.float32),
        ],
    )(x, w1p, asm, adm)


def _tc2_body(acc_ref, w2p_ref, b1p_ref, tile_ref, as2_ref, ad2_ref, s2_ref):
    a = acc_ref[0] + acc_ref[1]
    denom = a[:, 0:8]
    num = a[:, 16:80]
    dt = jnp.dot(denom, tile_ref[...], preferred_element_type=jnp.float32)
    out1 = jnp.maximum(num / (dt + 1e-16) + b1p_ref[...], 0.0)
    h2 = jnp.dot(out1, w2p_ref[...], preferred_element_type=jnp.float32)
    a_s = h2 * as2_ref[...]
    a_d = h2 * ad2_ref[...]
    z = jnp.zeros((a.shape[0], 13), jnp.float32)
    s2_ref[...] = jnp.concatenate([h2, a_s, a_d, z], axis=1)


def _tc2(acc1, w2p, b1p, tile8, a_src2, a_dst2, blk, grid):
    n = acc1.shape[1]
    return pl.pallas_call(
        _tc2_body,
        grid=(grid,),
        in_specs=[
            pl.BlockSpec((2, blk, 80), lambda i: (0, i, 0)),
            pl.BlockSpec((64, 1), lambda i: (0, 0)),
            pl.BlockSpec((1, 64), lambda i: (0, 0)),
            pl.BlockSpec((8, 64), lambda i: (0, 0)),
            pl.BlockSpec((1, 1), lambda i: (0, 0)),
            pl.BlockSpec((1, 1), lambda i: (0, 0)),
        ],
        out_specs=pl.BlockSpec((blk, 16), lambda i: (i, 0)),
        out_shape=jax.ShapeDtypeStruct((n, 16), jnp.float32),
    )(acc1, w2p, b1p, tile8, a_src2, a_dst2)


def _tc3_body(acc_ref, b2_ref, out_ref):
    a = acc_ref[0] + acc_ref[1]
    out_ref[...] = a[:, 0:1] / (a[:, 1:2] + 1e-16) + b2_ref[...]


def _tc3(acc2, b2, blk, grid):
    n = acc2.shape[1]
    return pl.pallas_call(
        _tc3_body,
        grid=(grid,),
        in_specs=[
            pl.BlockSpec((2, blk, 16), lambda i: (0, i, 0)),
            pl.BlockSpec((1, 1), lambda i: (0, 0)),
        ],
        out_specs=pl.BlockSpec((blk, 1), lambda i: (i, 0)),
        out_shape=jax.ShapeDtypeStruct((n, 1), jnp.float32),
    )(acc2, b2)


K1 = 128


def _make_sc1(n, e):
    n_chunks = e // K1
    rows_per_tile = n // NS
    npt = (n_chunks + NC * NS - 1) // (NC * NS)
    mesh = plsc.VectorSubcoreMesh(core_axis_name="c", subcore_axis_name="s")

    @functools.partial(
        pl.kernel,
        out_type=jax.ShapeDtypeStruct((NC, n, 80), jnp.float32),
        mesh=mesh,
        compiler_params=pltpu.CompilerParams(use_tc_tiling_on_sc=False),
        scratch_types=[
            pltpu.VMEM((npt, 128), jnp.int32),
            pltpu.VMEM((npt, 128), jnp.int32),
            pltpu.VMEM((2, K1, 80), jnp.float32),
            pltpu.VMEM((2, K1, 16), jnp.float32),
            pltpu.VMEM((2, K1, 80), jnp.float32),
            pltpu.VMEM((25, 80), jnp.float32),
            pltpu.VMEM_SHARED((n, 80), jnp.float32),
            pltpu.SemaphoreType.DMA((2,)),
            pltpu.SemaphoreType.DMA((2,)),
        ],
    )
    def sc1(s1_hbm, d1_hbm, src_hbm, dst_hbm, acc_hbm,
            src_i, dst_i, srow, drow, orow, zbuf, acc_sh, gsem, ssem):
        c = lax.axis_index("c")
        s = lax.axis_index("s")
        tile = c * NS + s
        base = tile * npt

        iota = lax.iota(jnp.int32, L)
        low8 = jnp.bitwise_and(iota, 7)
        zeros = jnp.zeros((L,), jnp.float32)

        def zz(i, _):
            for q in range(5):
                zbuf[i, pl.ds(q * 16, 16)] = zeros
            return 0
        lax.fori_loop(0, 25, zz, 0)

        for q in range(rows_per_tile // 25):
            pltpu.sync_copy(zbuf,
                            acc_sh.at[pl.ds(s * rows_per_tile + q * 25, 25)])

        pltpu.sync_copy(src_hbm.at[pl.ds(base, npt)], src_i)
        pltpu.sync_copy(dst_hbm.at[pl.ds(base, npt)], dst_i)
        plsc.subcore_barrier()

        def fire(j, b):
            @pl.when((j < npt) & (base + j < n_chunks))
            def _():
                pltpu.async_copy(s1_hbm.at[src_i.at[j]], srow.at[b],
                                 gsem.at[b])
                pltpu.async_copy(d1_hbm.at[dst_i.at[j]], drow.at[b],
                                 gsem.at[b])

        fire(0, 0)

        def chunk_loop(j, _):
            b = jnp.bitwise_and(j, 1)
            ok = base + j < n_chunks
            fire(j + 1, 1 - b)

            @pl.when((j >= 2) & (base + j - 2 < n_chunks))
            def _():
                pltpu.make_async_copy(orow.at[b],
                                      acc_sh.at[dst_i.at[j - 2]],
                                      ssem.at[b]).wait()

            @pl.when(ok)
            def _():
                pltpu.make_async_copy(s1_hbm.at[src_i.at[j]],
                                      srow.at[b], gsem.at[b]).wait()
                pltpu.make_async_copy(d1_hbm.at[dst_i.at[j]],
                                      drow.at[b], gsem.at[b]).wait()

                @plsc.parallel_loop(0, K1, 1, unroll=4)
                def edge(i):
                    v1 = srow[b, i, pl.ds(64, 16)]
                    v2 = drow[b, i, pl.ds(0, 16)]
                    al = v1 + v2
                    w = jnp.exp(jnp.maximum(al, 0.2 * al))
                    orow[b, i, pl.ds(0, 16)] = w
                    wd = w.at[low8].get(mode="promise_in_bounds")
                    for q in range(4):
                        orow[b, i, pl.ds(16 + q * 16, 16)] = (
                            srow[b, i, pl.ds(q * 16, 16)] * wd)

                pltpu.async_copy(orow.at[b], acc_sh.at[dst_i.at[j]],
                                 ssem.at[b], add=True)
            return 0
        lax.fori_loop(0, npt, chunk_loop, 0)

        for d in (2, 1):
            j = npt - d

            @pl.when(base + j < n_chunks)
            def _():
                pltpu.make_async_copy(orow.at[j % 2],
                                      acc_sh.at[dst_i.at[j]],
                                      ssem.at[j % 2]).wait()

        plsc.subcore_barrier()
        start = s * 624

        @pl.when(s < NS - 1)
        def _():
            pltpu.sync_copy(acc_sh.at[pl.ds(start, 624)],
                            acc_hbm.at[c, pl.ds(start, 624), :])

        @pl.when(s == NS - 1)
        def _():
            last = 624 * (NS - 1)
            pltpu.sync_copy(acc_sh.at[pl.ds(last, n - 624 * (NS - 1))],
                            acc_hbm.at[c, pl.ds(last, n - 624 * (NS - 1)), :])

    return sc1


K2 = 128


def _make_sc2(n, e):
    n_chunks = e // K2
    rows_per_tile = n // NS
    npt = (n_chunks + NC * NS - 1) // (NC * NS)
    mesh = plsc.VectorSubcoreMesh(core_axis_name="c", subcore_axis_name="s")

    @functools.partial(
        pl.kernel,
        out_type=jax.ShapeDtypeStruct((NC, n, 16), jnp.float32),
        mesh=mesh,
        compiler_params=pltpu.CompilerParams(use_tc_tiling_on_sc=False),
        scratch_types=[
            pltpu.VMEM((npt, 128), jnp.int32),
            pltpu.VMEM((npt, 128), jnp.int32),
            pltpu.VMEM((2, K2, 16), jnp.float32),
            pltpu.VMEM((2, K2, 16), jnp.float32),
            pltpu.VMEM((2, K2, 16), jnp.float32),
            pltpu.VMEM((25, 16), jnp.float32),
            pltpu.VMEM_SHARED((n, 16), jnp.float32),
            pltpu.SemaphoreType.DMA((2,)),
            pltpu.SemaphoreType.DMA((2,)),
        ],
    )
    def sc2(s2_hbm, src_hbm, dst_hbm, acc_hbm,
            src_i, dst_i, srow, drow, orow, zbuf, acc_sh, gsem, ssem):
        c = lax.axis_index("c")
        s = lax.axis_index("s")
        tile = c * NS + s
        base = tile * npt

        iota = lax.iota(jnp.int32, L)
        zeros = jnp.zeros((L,), jnp.float32)
        ones_i = jnp.full((L,), 1, jnp.int32)

        def zz(i, _):
            zbuf[i, pl.ds(0, 16)] = zeros
            return 0
        lax.fori_loop(0, 25, zz, 0)

        for q in range(rows_per_tile // 25):
            pltpu.sync_copy(zbuf,
                            acc_sh.at[pl.ds(s * rows_per_tile + q * 25, 25)])

        pltpu.sync_copy(src_hbm.at[pl.ds(base, npt)], src_i)
        pltpu.sync_copy(dst_hbm.at[pl.ds(base, npt)], dst_i)
        plsc.subcore_barrier()

        def fire(j, b):
            @pl.when((j < npt) & (base + j < n_chunks))
            def _():
                pltpu.async_copy(s2_hbm.at[src_i.at[j]], srow.at[b],
                                 gsem.at[b])
                pltpu.async_copy(s2_hbm.at[dst_i.at[j]], drow.at[b],
                                 gsem.at[b])

        fire(0, 0)

        def chunk_loop(j, _):
            b = jnp.bitwise_and(j, 1)
            ok = base + j < n_chunks
            fire(j + 1, 1 - b)

            @pl.when((j >= 2) & (base + j - 2 < n_chunks))
            def _():
                pltpu.make_async_copy(orow.at[b],
                                      acc_sh.at[dst_i.at[j - 2]],
                                      ssem.at[b]).wait()

            @pl.when(ok)
            def _():
                pltpu.make_async_copy(s2_hbm.at[src_i.at[j]],
                                      srow.at[b], gsem.at[b]).wait()
                pltpu.make_async_copy(s2_hbm.at[dst_i.at[j]],
                                      drow.at[b], gsem.at[b]).wait()

                @plsc.parallel_loop(0, K2, 1, unroll=8)
                def edge(i):
                    v_s = srow[b, i, pl.ds(0, 16)]
                    v_d = drow[b, i, pl.ds(0, 16)]
                    b_ad = v_d.at[ones_i + 1].get(mode="promise_in_bounds")
                    al = v_s + b_ad
                    wv = jnp.exp(jnp.maximum(al, 0.2 * al))
                    b_w = wv.at[ones_i].get(mode="promise_in_bounds")
                    prod = b_w * v_s
                    row = jnp.where(iota == 0, prod,
                                    jnp.where(iota == 1, b_w, zeros))
                    orow[b, i, pl.ds(0, 16)] = row

                pltpu.async_copy(orow.at[b], acc_sh.at[dst_i.at[j]],
                                 ssem.at[b], add=True)
            return 0
        lax.fori_loop(0, npt, chunk_loop, 0)

        for d in (2, 1):
            j = npt - d

            @pl.when(base + j < n_chunks)
            def _():
                pltpu.make_async_copy(orow.at[j % 2],
                                      acc_sh.at[dst_i.at[j]],
                                      ssem.at[j % 2]).wait()

        plsc.subcore_barrier()
        start = s * 624

        @pl.when(s < NS - 1)
        def _():
            pltpu.sync_copy(acc_sh.at[pl.ds(start, 624)],
                            acc_hbm.at[c, pl.ds(start, 624), :])

        @pl.when(s == NS - 1)
        def _():
            last = 624 * (NS - 1)
            pltpu.sync_copy(acc_sh.at[pl.ds(last, n - 624 * (NS - 1))],
                            acc_hbm.at[c, pl.ds(last, n - 624 * (NS - 1)), :])

    return sc2


def kernel(x, edge_index, W1, a_src1, a_dst1, b1, W2, a_src2, a_dst2, b2):
    n = x.shape[0]
    e = edge_index.shape[1]

    j = jnp.arange(64)
    perm = (j % 8) * 8 + j // 8
    w1p = W1[:, perm]
    w2p = W2[perm, :]
    b1p = b1[perm][None, :]
    eye8 = jnp.eye(8, dtype=jnp.float32)
    asm = (a_src1.T[:, :, None] * eye8[None, :, :]).reshape(64, 8)
    adm = (a_dst1.T[:, :, None] * eye8[None, :, :]).reshape(64, 8)
    tile8 = jnp.tile(eye8, (1, 8))

    n_chunks = e // 128
    npt = (n_chunks + NC * NS - 1) // (NC * NS)
    pad_rows = npt * NC * NS - n_chunks
    src_r = edge_index[0].reshape(n_chunks, 128)
    dst_r = edge_index[1].reshape(n_chunks, 128)
    if pad_rows:
        zpad = jnp.zeros((pad_rows, 128), jnp.int32)
        src_r = jnp.concatenate([src_r, zpad], axis=0)
        dst_r = jnp.concatenate([dst_r, zpad], axis=0)

    blk, grid = 1000, n // 1000

    s1, d1 = _tc1(x, w1p, asm, adm, blk, grid)
    acc1 = _make_sc1(n, e)(s1, d1, src_r, dst_r)
    s2 = _tc2(acc1, w2p, b1p, tile8, a_src2, a_dst2, blk, grid)
    acc2 = _make_sc2(n, e)(s2, src_r, dst_r)
    out = _tc3(acc2, b2.reshape(1, 1), blk, grid)
    return out[:, 0]


# speedup vs baseline: 202.5324x; 1.0247x over previous
import functools
import jax
import jax.numpy as jnp
from jax import lax
from jax.experimental import pallas as pl
from jax.experimental.pallas import tpu as pltpu
from jax.experimental.pallas import tpu_sc as plsc

NC = 2
NS = 16
L = 16


def _tc1_body(x_ref, w1p_ref, asm_ref, adm_ref, s1_ref, d1_ref):
    h = jnp.dot(x_ref[...], w1p_ref[...], preferred_element_type=jnp.float32)
    a_s = jnp.dot(h, asm_ref[...], preferred_element_type=jnp.float32)
    a_d = jnp.dot(h, adm_ref[...], preferred_element_type=jnp.float32)
    s1_ref[...] = jnp.concatenate([h, a_s, a_d], axis=1)
    d1_ref[...] = jnp.concatenate([a_d, jnp.zeros_like(a_d)], axis=1)


def _tc1(x, w1p, asm, adm, blk, grid):
    n = x.shape[0]
    f_in = x.shape[1]
    return pl.pallas_call(
        _tc1_body,
        grid=(grid,),
        in_specs=[
            pl.BlockSpec((blk, f_in), lambda i: (i, 0)),
            pl.BlockSpec((f_in, 64), lambda i: (0, 0)),
            pl.BlockSpec((64, 8), lambda i: (0, 0)),
            pl.BlockSpec((64, 8), lambda i: (0, 0)),
        ],
        out_specs=[
            pl.BlockSpec((blk, 80), lambda i: (i, 0)),
            pl.BlockSpec((blk, 16), lambda i: (i, 0)),
        ],
        out_shape=[
            jax.ShapeDtypeStruct((n, 80), jnp.float32),
            jax.ShapeDtypeStruct((n, 16), jnp.float32),
        ],
    )(x, w1p, asm, adm)


def _tc2_body(acc_ref, w2p_ref, b1p_ref, tile_ref, as2_ref, ad2_ref, s2_ref):
    a = acc_ref[0] + acc_ref[1]
    denom = a[:, 0:8]
    num = a[:, 16:80]
    dt = jnp.dot(denom, tile_ref[...], preferred_element_type=jnp.float32)
    out1 = jnp.maximum(num / (dt + 1e-16) + b1p_ref[...], 0.0)
    h2 = jnp.dot(out1, w2p_ref[...], preferred_element_type=jnp.float32)
    a_s = h2 * as2_ref[...]
    a_d = h2 * ad2_ref[...]
    z = jnp.zeros((a.shape[0], 13), jnp.float32)
    s2_ref[...] = jnp.concatenate([h2, a_s, a_d, z], axis=1)


def _tc2(acc1, w2p, b1p, tile8, a_src2, a_dst2, blk, grid):
    n = acc1.shape[1]
    return pl.pallas_call(
        _tc2_body,
        grid=(grid,),
        in_specs=[
            pl.BlockSpec((2, blk, 80), lambda i: (0, i, 0)),
            pl.BlockSpec((64, 1), lambda i: (0, 0)),
            pl.BlockSpec((1, 64), lambda i: (0, 0)),
            pl.BlockSpec((8, 64), lambda i: (0, 0)),
            pl.BlockSpec((1, 1), lambda i: (0, 0)),
            pl.BlockSpec((1, 1), lambda i: (0, 0)),
        ],
        out_specs=pl.BlockSpec((blk, 16), lambda i: (i, 0)),
        out_shape=jax.ShapeDtypeStruct((n, 16), jnp.float32),
    )(acc1, w2p, b1p, tile8, a_src2, a_dst2)


def _tc3_body(acc_ref, b2_ref, out_ref):
    a = acc_ref[0] + acc_ref[1]
    out_ref[...] = a[:, 0:1] / (a[:, 1:2] + 1e-16) + b2_ref[...]


def _tc3(acc2, b2, blk, grid):
    n = acc2.shape[1]
    return pl.pallas_call(
        _tc3_body,
        grid=(grid,),
        in_specs=[
            pl.BlockSpec((2, blk, 16), lambda i: (0, i, 0)),
            pl.BlockSpec((1, 1), lambda i: (0, 0)),
        ],
        out_specs=pl.BlockSpec((blk, 1), lambda i: (i, 0)),
        out_shape=jax.ShapeDtypeStruct((n, 1), jnp.float32),
    )(acc2, b2)


K1 = 128


def _make_sc1(n, e):
    n_chunks = e // K1
    rows_per_tile = n // NS
    npt = (n_chunks + NC * NS - 1) // (NC * NS)
    mesh = plsc.VectorSubcoreMesh(core_axis_name="c", subcore_axis_name="s")

    @functools.partial(
        pl.kernel,
        out_type=jax.ShapeDtypeStruct((NC, n, 80), jnp.float32),
        mesh=mesh,
        compiler_params=pltpu.CompilerParams(use_tc_tiling_on_sc=False),
        scratch_types=[
            pltpu.VMEM((npt, 128), jnp.int32),
            pltpu.VMEM((npt, 128), jnp.int32),
            pltpu.VMEM((2, K1, 80), jnp.float32),
            pltpu.VMEM((2, K1, 16), jnp.float32),
            pltpu.VMEM((2, K1, 80), jnp.float32),
            pltpu.VMEM((25, 80), jnp.float32),
            pltpu.VMEM_SHARED((n, 80), jnp.float32),
            pltpu.SemaphoreType.DMA((2,)),
            pltpu.SemaphoreType.DMA((2,)),
        ],
    )
    def sc1(s1_hbm, d1_hbm, src_hbm, dst_hbm, acc_hbm,
            src_i, dst_i, srow, drow, orow, zbuf, acc_sh, gsem, ssem):
        c = lax.axis_index("c")
        s = lax.axis_index("s")
        tile = c * NS + s
        base = tile * npt

        iota = lax.iota(jnp.int32, L)
        low8 = jnp.bitwise_and(iota, 7)
        zeros = jnp.zeros((L,), jnp.float32)

        def zz(i, _):
            for q in range(5):
                zbuf[i, pl.ds(q * 16, 16)] = zeros
            return 0
        lax.fori_loop(0, 25, zz, 0)

        idx_cps = [pltpu.async_copy(src_hbm.at[pl.ds(base, npt)], src_i,
                                    gsem.at[0]),
                   pltpu.async_copy(dst_hbm.at[pl.ds(base, npt)], dst_i,
                                    gsem.at[0])]
        zero_cps = [
            pltpu.async_copy(
                zbuf, acc_sh.at[pl.ds(s * rows_per_tile + q * 25, 25)],
                gsem.at[1])
            for q in range(rows_per_tile // 25)]
        for cp in idx_cps + zero_cps:
            cp.wait()
        plsc.subcore_barrier()

        def fire(j, b):
            @pl.when((j < npt) & (base + j < n_chunks))
            def _():
                pltpu.async_copy(s1_hbm.at[src_i.at[j]], srow.at[b],
                                 gsem.at[b])
                pltpu.async_copy(d1_hbm.at[dst_i.at[j]], drow.at[b],
                                 gsem.at[b])

        fire(0, 0)

        def chunk_loop(j, _):
            b = jnp.bitwise_and(j, 1)
            ok = base + j < n_chunks
            fire(j + 1, 1 - b)

            @pl.when((j >= 2) & (base + j - 2 < n_chunks))
            def _():
                pltpu.make_async_copy(orow.at[b],
                                      acc_sh.at[dst_i.at[j - 2]],
                                      ssem.at[b]).wait()

            @pl.when(ok)
            def _():
                pltpu.make_async_copy(s1_hbm.at[src_i.at[j]],
                                      srow.at[b], gsem.at[b]).wait()
                pltpu.make_async_copy(d1_hbm.at[dst_i.at[j]],
                                      drow.at[b], gsem.at[b]).wait()

                @plsc.parallel_loop(0, K1, 1, unroll=4)
                def edge(i):
                    v1 = srow[b, i, pl.ds(64, 16)]
                    v2 = drow[b, i, pl.ds(0, 16)]
                    al = v1 + v2
                    w = jnp.exp(jnp.maximum(al, 0.2 * al))
                    orow[b, i, pl.ds(0, 16)] = w
                    wd = w.at[low8].get(mode="promise_in_bounds")
                    for q in range(4):
                        orow[b, i, pl.ds(16 + q * 16, 16)] = (
                            srow[b, i, pl.ds(q * 16, 16)] * wd)

                pltpu.async_copy(orow.at[b], acc_sh.at[dst_i.at[j]],
                                 ssem.at[b], add=True)
            return 0
        lax.fori_loop(0, npt, chunk_loop, 0)

        for d in (2, 1):
            j = npt - d

            @pl.when(base + j < n_chunks)
            def _():
                pltpu.make_async_copy(orow.at[j % 2],
                                      acc_sh.at[dst_i.at[j]],
                                      ssem.at[j % 2]).wait()

        plsc.subcore_barrier()
        start = s * 624

        @pl.when(s < NS - 1)
        def _():
            pltpu.sync_copy(acc_sh.at[pl.ds(start, 624)],
                            acc_hbm.at[c, pl.ds(start, 624), :])

        @pl.when(s == NS - 1)
        def _():
            last = 624 * (NS - 1)
            pltpu.sync_copy(acc_sh.at[pl.ds(last, n - 624 * (NS - 1))],
                            acc_hbm.at[c, pl.ds(last, n - 624 * (NS - 1)), :])

    return sc1


K2 = 128


def _make_sc2(n, e):
    n_chunks = e // K2
    rows_per_tile = n // NS
    npt = (n_chunks + NC * NS - 1) // (NC * NS)
    mesh = plsc.VectorSubcoreMesh(core_axis_name="c", subcore_axis_name="s")

    @functools.partial(
        pl.kernel,
        out_type=jax.ShapeDtypeStruct((NC, n, 16), jnp.float32),
        mesh=mesh,
        compiler_params=pltpu.CompilerParams(use_tc_tiling_on_sc=False),
        scratch_types=[
            pltpu.VMEM((npt, 128), jnp.int32),
            pltpu.VMEM((npt, 128), jnp.int32),
            pltpu.VMEM((2, K2, 16), jnp.float32),
            pltpu.VMEM((2, K2, 16), jnp.float32),
            pltpu.VMEM((2, K2, 16), jnp.float32),
            pltpu.VMEM((25, 16), jnp.float32),
            pltpu.VMEM_SHARED((n, 16), jnp.float32),
            pltpu.SemaphoreType.DMA((2,)),
            pltpu.SemaphoreType.DMA((2,)),
        ],
    )
    def sc2(s2_hbm, src_hbm, dst_hbm, acc_hbm,
            src_i, dst_i, srow, drow, orow, zbuf, acc_sh, gsem, ssem):
        c = lax.axis_index("c")
        s = lax.axis_index("s")
        tile = c * NS + s
        base = tile * npt

        iota = lax.iota(jnp.int32, L)
        zeros = jnp.zeros((L,), jnp.float32)
        ones_i = jnp.full((L,), 1, jnp.int32)

        def zz(i, _):
            zbuf[i, pl.ds(0, 16)] = zeros
            return 0
        lax.fori_loop(0, 25, zz, 0)

        idx_cps = [pltpu.async_copy(src_hbm.at[pl.ds(base, npt)], src_i,
                                    gsem.at[0]),
                   pltpu.async_copy(dst_hbm.at[pl.ds(base, npt)], dst_i,
                                    gsem.at[0])]
        zero_cps = [
            pltpu.async_copy(
                zbuf, acc_sh.at[pl.ds(s * rows_per_tile + q * 25, 25)],
                gsem.at[1])
            for q in range(rows_per_tile // 25)]
        for cp in idx_cps + zero_cps:
            cp.wait()
        plsc.subcore_barrier()

        def fire(j, b):
            @pl.when((j < npt) & (base + j < n_chunks))
            def _():
                pltpu.async_copy(s2_hbm.at[src_i.at[j]], srow.at[b],
                                 gsem.at[b])
                pltpu.async_copy(s2_hbm.at[dst_i.at[j]], drow.at[b],
                                 gsem.at[b])

        fire(0, 0)

        def chunk_loop(j, _):
            b = jnp.bitwise_and(j, 1)
            ok = base + j < n_chunks
            fire(j + 1, 1 - b)

            @pl.when((j >= 2) & (base + j - 2 < n_chunks))
            def _():
                pltpu.make_async_copy(orow.at[b],
                                      acc_sh.at[dst_i.at[j - 2]],
                                      ssem.at[b]).wait()

            @pl.when(ok)
            def _():
                pltpu.make_async_copy(s2_hbm.at[src_i.at[j]],
                                      srow.at[b], gsem.at[b]).wait()
                pltpu.make_async_copy(s2_hbm.at[dst_i.at[j]],
                                      drow.at[b], gsem.at[b]).wait()

                @plsc.parallel_loop(0, K2, 1, unroll=8)
                def edge(i):
                    v_s = srow[b, i, pl.ds(0, 16)]
                    v_d = drow[b, i, pl.ds(0, 16)]
                    b_ad = v_d.at[ones_i + 1].get(mode="promise_in_bounds")
                    al = v_s + b_ad
                    wv = jnp.exp(jnp.maximum(al, 0.2 * al))
                    b_w = wv.at[ones_i].get(mode="promise_in_bounds")
                    prod = b_w * v_s
                    row = jnp.where(iota == 0, prod,
                                    jnp.where(iota == 1, b_w, zeros))
                    orow[b, i, pl.ds(0, 16)] = row

                pltpu.async_copy(orow.at[b], acc_sh.at[dst_i.at[j]],
                                 ssem.at[b], add=True)
            return 0
        lax.fori_loop(0, npt, chunk_loop, 0)

        for d in (2, 1):
            j = npt - d

            @pl.when(base + j < n_chunks)
            def _():
                pltpu.make_async_copy(orow.at[j % 2],
                                      acc_sh.at[dst_i.at[j]],
                                      ssem.at[j % 2]).wait()

        plsc.subcore_barrier()
        start = s * 624

        @pl.when(s < NS - 1)
        def _():
            pltpu.sync_copy(acc_sh.at[pl.ds(start, 624)],
                            acc_hbm.at[c, pl.ds(start, 624), :])

        @pl.when(s == NS - 1)
        def _():
            last = 624 * (NS - 1)
            pltpu.sync_copy(acc_sh.at[pl.ds(last, n - 624 * (NS - 1))],
                            acc_hbm.at[c, pl.ds(last, n - 624 * (NS - 1)), :])

    return sc2


def kernel(x, edge_index, W1, a_src1, a_dst1, b1, W2, a_src2, a_dst2, b2):
    n = x.shape[0]
    e = edge_index.shape[1]

    j = jnp.arange(64)
    perm = (j % 8) * 8 + j // 8
    w1p = W1[:, perm]
    w2p = W2[perm, :]
    b1p = b1[perm][None, :]
    eye8 = jnp.eye(8, dtype=jnp.float32)
    asm = (a_src1.T[:, :, None] * eye8[None, :, :]).reshape(64, 8)
    adm = (a_dst1.T[:, :, None] * eye8[None, :, :]).reshape(64, 8)
    tile8 = jnp.tile(eye8, (1, 8))

    n_chunks = e // 128
    npt = (n_chunks + NC * NS - 1) // (NC * NS)
    pad_rows = npt * NC * NS - n_chunks
    src_r = edge_index[0].reshape(n_chunks, 128)
    dst_r = edge_index[1].reshape(n_chunks, 128)
    if pad_rows:
        zpad = jnp.zeros((pad_rows, 128), jnp.int32)
        src_r = jnp.concatenate([src_r, zpad], axis=0)
        dst_r = jnp.concatenate([dst_r, zpad], axis=0)

    blk, grid = 1000, n // 1000

    s1, d1 = _tc1(x, w1p, asm, adm, blk, grid)
    acc1 = _make_sc1(n, e)(s1, d1, src_r, dst_r)
    s2 = _tc2(acc1, w2p, b1p, tile8, a_src2, a_dst2, blk, grid)
    acc2 = _make_sc2(n, e)(s2, src_r, dst_r)
    out = _tc3(acc2, b2.reshape(1, 1), blk, grid)
    return out[:, 0]


# speedup vs baseline: 218.6449x; 1.0796x over previous
import functools
import jax
import jax.numpy as jnp
from jax import lax
from jax.experimental import pallas as pl
from jax.experimental.pallas import tpu as pltpu
from jax.experimental.pallas import tpu_sc as plsc

NC = 2
NS = 16
L = 16


def _tc1_body(x_ref, w1p_ref, asm_ref, adm_ref, s1_ref, d1_ref):
    h = jnp.dot(x_ref[...], w1p_ref[...], preferred_element_type=jnp.float32)
    a_s = jnp.dot(h, asm_ref[...], preferred_element_type=jnp.float32)
    a_d = jnp.dot(h, adm_ref[...], preferred_element_type=jnp.float32)
    s1_ref[...] = jnp.concatenate([h, a_s, a_d], axis=1)
    d1_ref[...] = jnp.concatenate([a_d, jnp.zeros_like(a_d)], axis=1)


def _tc1(x, w1p, asm, adm, blk, grid):
    n = x.shape[0]
    f_in = x.shape[1]
    return pl.pallas_call(
        _tc1_body,
        grid=(grid,),
        in_specs=[
            pl.BlockSpec((blk, f_in), lambda i: (i, 0)),
            pl.BlockSpec((f_in, 64), lambda i: (0, 0)),
            pl.BlockSpec((64, 8), lambda i: (0, 0)),
            pl.BlockSpec((64, 8), lambda i: (0, 0)),
        ],
        out_specs=[
            pl.BlockSpec((blk, 80), lambda i: (i, 0)),
            pl.BlockSpec((blk, 16), lambda i: (i, 0)),
        ],
        out_shape=[
            jax.ShapeDtypeStruct((n, 80), jnp.float32),
            jax.ShapeDtypeStruct((n, 16), jnp.float32),
        ],
    )(x, w1p, asm, adm)


def _tc2_body(acc_ref, w2p_ref, b1p_ref, tile_ref, as2_ref, ad2_ref, s2_ref):
    a = acc_ref[0] + acc_ref[1]
    denom = a[:, 0:8]
    num = a[:, 16:80]
    dt = jnp.dot(denom, tile_ref[...], preferred_element_type=jnp.float32)
    out1 = jnp.maximum(num / (dt + 1e-16) + b1p_ref[...], 0.0)
    h2 = jnp.dot(out1, w2p_ref[...], preferred_element_type=jnp.float32)
    a_s = h2 * as2_ref[...]
    a_d = h2 * ad2_ref[...]
    z = jnp.zeros((a.shape[0], 13), jnp.float32)
    s2_ref[...] = jnp.concatenate([h2, a_s, a_d, z], axis=1)


def _tc2(acc1, w2p, b1p, tile8, a_src2, a_dst2, blk, grid):
    n = acc1.shape[1]
    return pl.pallas_call(
        _tc2_body,
        grid=(grid,),
        in_specs=[
            pl.BlockSpec((2, blk, 128), lambda i: (0, i, 0)),
            pl.BlockSpec((64, 1), lambda i: (0, 0)),
            pl.BlockSpec((1, 64), lambda i: (0, 0)),
            pl.BlockSpec((8, 64), lambda i: (0, 0)),
            pl.BlockSpec((1, 1), lambda i: (0, 0)),
            pl.BlockSpec((1, 1), lambda i: (0, 0)),
        ],
        out_specs=pl.BlockSpec((blk, 16), lambda i: (i, 0)),
        out_shape=jax.ShapeDtypeStruct((n, 16), jnp.float32),
    )(acc1, w2p, b1p, tile8, a_src2, a_dst2)


def _tc3_body(acc_ref, b2_ref, out_ref):
    a = acc_ref[0] + acc_ref[1]
    out_ref[...] = a[:, 0:1] / (a[:, 1:2] + 1e-16) + b2_ref[...]


def _tc3(acc2, b2, blk, grid):
    n = acc2.shape[1]
    return pl.pallas_call(
        _tc3_body,
        grid=(grid,),
        in_specs=[
            pl.BlockSpec((2, blk, 16), lambda i: (0, i, 0)),
            pl.BlockSpec((1, 1), lambda i: (0, 0)),
        ],
        out_specs=pl.BlockSpec((blk, 1), lambda i: (i, 0)),
        out_shape=jax.ShapeDtypeStruct((n, 1), jnp.float32),
    )(acc2, b2)


K1 = 128


def _make_sc1(n, e):
    n_chunks = e // K1
    rows_per_tile = n // NS
    npt = (n_chunks + NC * NS - 1) // (NC * NS)
    mesh = plsc.VectorSubcoreMesh(core_axis_name="c", subcore_axis_name="s")

    @functools.partial(
        pl.kernel,
        out_type=jax.ShapeDtypeStruct((NC, n, 128), jnp.float32),
        mesh=mesh,
        compiler_params=pltpu.CompilerParams(use_tc_tiling_on_sc=False),
        scratch_types=[
            pltpu.VMEM((npt, 128), jnp.int32),
            pltpu.VMEM((npt, 128), jnp.int32),
            pltpu.VMEM((2, K1, 80), jnp.float32),
            pltpu.VMEM((2, K1, 16), jnp.float32),
            pltpu.VMEM((2, K1, 80), jnp.float32),
            pltpu.VMEM((25, 80), jnp.float32),
            pltpu.VMEM_SHARED((n, 80), jnp.float32),
            pltpu.SemaphoreType.DMA((2,)),
            pltpu.SemaphoreType.DMA((2,)),
        ],
    )
    def sc1(s1_hbm, d1_hbm, src_hbm, dst_hbm, acc_hbm,
            src_i, dst_i, srow, drow, orow, zbuf, acc_sh, gsem, ssem):
        c = lax.axis_index("c")
        s = lax.axis_index("s")
        tile = c * NS + s
        base = tile * npt

        iota = lax.iota(jnp.int32, L)
        low8 = jnp.bitwise_and(iota, 7)
        zeros = jnp.zeros((L,), jnp.float32)

        def zz(i, _):
            for q in range(5):
                zbuf[i, pl.ds(q * 16, 16)] = zeros
            return 0
        lax.fori_loop(0, 25, zz, 0)

        idx_cps = [pltpu.async_copy(src_hbm.at[pl.ds(base, npt)], src_i,
                                    gsem.at[0]),
                   pltpu.async_copy(dst_hbm.at[pl.ds(base, npt)], dst_i,
                                    gsem.at[0])]
        zero_cps = [
            pltpu.async_copy(
                zbuf, acc_sh.at[pl.ds(s * rows_per_tile + q * 25, 25)],
                gsem.at[1])
            for q in range(rows_per_tile // 25)]
        for cp in idx_cps + zero_cps:
            cp.wait()
        plsc.subcore_barrier()

        def fire(j, b):
            @pl.when((j < npt) & (base + j < n_chunks))
            def _():
                pltpu.async_copy(s1_hbm.at[src_i.at[j]], srow.at[b],
                                 gsem.at[b])
                pltpu.async_copy(d1_hbm.at[dst_i.at[j]], drow.at[b],
                                 gsem.at[b])

        fire(0, 0)

        def chunk_loop(j, _):
            b = jnp.bitwise_and(j, 1)
            ok = base + j < n_chunks
            fire(j + 1, 1 - b)

            @pl.when((j >= 2) & (base + j - 2 < n_chunks))
            def _():
                pltpu.make_async_copy(orow.at[b],
                                      acc_sh.at[dst_i.at[j - 2]],
                                      ssem.at[b]).wait()

            @pl.when(ok)
            def _():
                pltpu.make_async_copy(s1_hbm.at[src_i.at[j]],
                                      srow.at[b], gsem.at[b]).wait()
                pltpu.make_async_copy(d1_hbm.at[dst_i.at[j]],
                                      drow.at[b], gsem.at[b]).wait()

                @plsc.parallel_loop(0, K1, 1, unroll=4)
                def edge(i):
                    v1 = srow[b, i, pl.ds(64, 16)]
                    v2 = drow[b, i, pl.ds(0, 16)]
                    al = v1 + v2
                    w = jnp.exp(jnp.maximum(al, 0.2 * al))
                    orow[b, i, pl.ds(0, 16)] = w
                    wd = w.at[low8].get(mode="promise_in_bounds")
                    for q in range(4):
                        orow[b, i, pl.ds(16 + q * 16, 16)] = (
                            srow[b, i, pl.ds(q * 16, 16)] * wd)

                pltpu.async_copy(orow.at[b], acc_sh.at[dst_i.at[j]],
                                 ssem.at[b], add=True)
            return 0
        lax.fori_loop(0, npt, chunk_loop, 0)

        for d in (2, 1):
            j = npt - d

            @pl.when(base + j < n_chunks)
            def _():
                pltpu.make_async_copy(orow.at[j % 2],
                                      acc_sh.at[dst_i.at[j]],
                                      ssem.at[j % 2]).wait()

        plsc.subcore_barrier()
        start = s * 624

        @pl.when(s < NS - 1)
        def _():
            pltpu.sync_copy(acc_sh.at[pl.ds(start, 624)],
                            acc_hbm.at[c, pl.ds(start, 624), pl.ds(0, 80)])

        @pl.when(s == NS - 1)
        def _():
            last = 624 * (NS - 1)
            pltpu.sync_copy(acc_sh.at[pl.ds(last, n - 624 * (NS - 1))],
                            acc_hbm.at[c, pl.ds(last, n - 624 * (NS - 1)),
                                       pl.ds(0, 80)])

    return sc1


K2 = 128


def _make_sc2(n, e):
    n_chunks = e // K2
    rows_per_tile = n // NS
    npt = (n_chunks + NC * NS - 1) // (NC * NS)
    mesh = plsc.VectorSubcoreMesh(core_axis_name="c", subcore_axis_name="s")

    @functools.partial(
        pl.kernel,
        out_type=jax.ShapeDtypeStruct((NC, n, 16), jnp.float32),
        mesh=mesh,
        compiler_params=pltpu.CompilerParams(use_tc_tiling_on_sc=False),
        scratch_types=[
            pltpu.VMEM((npt, 128), jnp.int32),
            pltpu.VMEM((npt, 128), jnp.int32),
            pltpu.VMEM((2, K2, 16), jnp.float32),
            pltpu.VMEM((2, K2, 16), jnp.float32),
            pltpu.VMEM((2, K2, 16), jnp.float32),
            pltpu.VMEM((25, 16), jnp.float32),
            pltpu.VMEM_SHARED((n, 16), jnp.float32),
            pltpu.SemaphoreType.DMA((2,)),
            pltpu.SemaphoreType.DMA((2,)),
        ],
    )
    def sc2(s2_hbm, src_hbm, dst_hbm, acc_hbm,
            src_i, dst_i, srow, drow, orow, zbuf, acc_sh, gsem, ssem):
        c = lax.axis_index("c")
        s = lax.axis_index("s")
        tile = c * NS + s
        base = tile * npt

        iota = lax.iota(jnp.int32, L)
        zeros = jnp.zeros((L,), jnp.float32)
        ones_i = jnp.full((L,), 1, jnp.int32)

        def zz(i, _):
            zbuf[i, pl.ds(0, 16)] = zeros
            return 0
        lax.fori_loop(0, 25, zz, 0)

        idx_cps = [pltpu.async_copy(src_hbm.at[pl.ds(base, npt)], src_i,
                                    gsem.at[0]),
                   pltpu.async_copy(dst_hbm.at[pl.ds(base, npt)], dst_i,
                                    gsem.at[0])]
        zero_cps = [
            pltpu.async_copy(
                zbuf, acc_sh.at[pl.ds(s * rows_per_tile + q * 25, 25)],
                gsem.at[1])
            for q in range(rows_per_tile // 25)]
        for cp in idx_cps + zero_cps:
            cp.wait()
        plsc.subcore_barrier()

        def fire(j, b):
            @pl.when((j < npt) & (base + j < n_chunks))
            def _():
                pltpu.async_copy(s2_hbm.at[src_i.at[j]], srow.at[b],
                                 gsem.at[b])
                pltpu.async_copy(s2_hbm.at[dst_i.at[j]], drow.at[b],
                                 gsem.at[b])

        fire(0, 0)

        def chunk_loop(j, _):
            b = jnp.bitwise_and(j, 1)
            ok = base + j < n_chunks
            fire(j + 1, 1 - b)

            @pl.when((j >= 2) & (base + j - 2 < n_chunks))
            def _():
                pltpu.make_async_copy(orow.at[b],
                                      acc_sh.at[dst_i.at[j - 2]],
                                      ssem.at[b]).wait()

            @pl.when(ok)
            def _():
                pltpu.make_async_copy(s2_hbm.at[src_i.at[j]],
                                      srow.at[b], gsem.at[b]).wait()
                pltpu.make_async_copy(s2_hbm.at[dst_i.at[j]],
                                      drow.at[b], gsem.at[b]).wait()

                @plsc.parallel_loop(0, K2, 1, unroll=8)
                def edge(i):
                    v_s = srow[b, i, pl.ds(0, 16)]
                    v_d = drow[b, i, pl.ds(0, 16)]
                    b_ad = v_d.at[ones_i + 1].get(mode="promise_in_bounds")
                    al = v_s + b_ad
                    wv = jnp.exp(jnp.maximum(al, 0.2 * al))
                    b_w = wv.at[ones_i].get(mode="promise_in_bounds")
                    prod = b_w * v_s
                    row = jnp.where(iota == 0, prod,
                                    jnp.where(iota == 1, b_w, zeros))
                    orow[b, i, pl.ds(0, 16)] = row

                pltpu.async_copy(orow.at[b], acc_sh.at[dst_i.at[j]],
                                 ssem.at[b], add=True)
            return 0
        lax.fori_loop(0, npt, chunk_loop, 0)

        for d in (2, 1):
            j = npt - d

            @pl.when(base + j < n_chunks)
            def _():
                pltpu.make_async_copy(orow.at[j % 2],
                                      acc_sh.at[dst_i.at[j]],
                                      ssem.at[j % 2]).wait()

        plsc.subcore_barrier()
        start = s * 624

        @pl.when(s < NS - 1)
        def _():
            pltpu.sync_copy(acc_sh.at[pl.ds(start, 624)],
                            acc_hbm.at[c, pl.ds(start, 624), :])

        @pl.when(s == NS - 1)
        def _():
            last = 624 * (NS - 1)
            pltpu.sync_copy(acc_sh.at[pl.ds(last, n - 624 * (NS - 1))],
                            acc_hbm.at[c, pl.ds(last, n - 624 * (NS - 1)), :])

    return sc2


def kernel(x, edge_index, W1, a_src1, a_dst1, b1, W2, a_src2, a_dst2, b2):
    n = x.shape[0]
    e = edge_index.shape[1]

    j = jnp.arange(64)
    perm = (j % 8) * 8 + j // 8
    w1p = W1[:, perm]
    w2p = W2[perm, :]
    b1p = b1[perm][None, :]
    eye8 = jnp.eye(8, dtype=jnp.float32)
    asm = (a_src1.T[:, :, None] * eye8[None, :, :]).reshape(64, 8)
    adm = (a_dst1.T[:, :, None] * eye8[None, :, :]).reshape(64, 8)
    tile8 = jnp.tile(eye8, (1, 8))

    n_chunks = e // 128
    npt = (n_chunks + NC * NS - 1) // (NC * NS)
    pad_rows = npt * NC * NS - n_chunks
    ei_p = lax.pad(edge_index, jnp.int32(0),
                   ((0, 0, 0), (0, pad_rows * 128, 0)))
    ei_p = ei_p.reshape(2, n_chunks + pad_rows, 128)
    src_r = ei_p[0]
    dst_r = ei_p[1]

    blk, grid = 1000, n // 1000

    s1, d1 = _tc1(x, w1p, asm, adm, blk, grid)
    acc1 = _make_sc1(n, e)(s1, d1, src_r, dst_r)
    s2 = _tc2(acc1, w2p, b1p, tile8, a_src2, a_dst2, blk, grid)
    acc2 = _make_sc2(n, e)(s2, src_r, dst_r)
    out = _tc3(acc2, b2.reshape(1, 1), blk, grid)
    return out[:, 0]


# speedup vs baseline: 234.2643x; 1.0714x over previous
import functools
import jax
import jax.numpy as jnp
from jax import lax
from jax.experimental import pallas as pl
from jax.experimental.pallas import tpu as pltpu
from jax.experimental.pallas import tpu_sc as plsc

NC = 2
NS = 16
L = 16


def _tc1_body(x_ref, w1p_ref, asm_ref, adm_ref, s1_ref, d1_ref):
    h = jnp.dot(x_ref[...], w1p_ref[...], preferred_element_type=jnp.float32)
    a_s = jnp.dot(h, asm_ref[...], preferred_element_type=jnp.float32)
    a_d = jnp.dot(h, adm_ref[...], preferred_element_type=jnp.float32)
    s1_ref[...] = jnp.concatenate([h, a_s, a_d], axis=1)
    d1_ref[...] = jnp.concatenate([a_d, jnp.zeros_like(a_d)], axis=1)


def _tc1(x, w1p, asm, adm, blk, grid):
    n = x.shape[0]
    f_in = x.shape[1]
    return pl.pallas_call(
        _tc1_body,
        grid=(grid,),
        in_specs=[
            pl.BlockSpec((blk, f_in), lambda i: (i, 0)),
            pl.BlockSpec((f_in, 64), lambda i: (0, 0)),
            pl.BlockSpec((64, 8), lambda i: (0, 0)),
            pl.BlockSpec((64, 8), lambda i: (0, 0)),
        ],
        out_specs=[
            pl.BlockSpec((blk, 80), lambda i: (i, 0)),
            pl.BlockSpec((blk, 16), lambda i: (i, 0)),
        ],
        out_shape=[
            jax.ShapeDtypeStruct((n, 80), jnp.float32),
            jax.ShapeDtypeStruct((n, 16), jnp.float32),
        ],
    )(x, w1p, asm, adm)


def _tc2_body(acc_ref, w2p_ref, b1p_ref, tile_ref, as2_ref, ad2_ref, s2_ref):
    a = acc_ref[0] + acc_ref[1]
    denom = a[:, 0:8]
    num = a[:, 16:80]
    dt = jnp.dot(denom, tile_ref[...], preferred_element_type=jnp.float32)
    out1 = jnp.maximum(num / (dt + 1e-16) + b1p_ref[...], 0.0)
    h2 = jnp.dot(out1, w2p_ref[...], preferred_element_type=jnp.float32)
    a_s = h2 * as2_ref[...]
    a_d = h2 * ad2_ref[...]
    z = jnp.zeros((a.shape[0], 13), jnp.float32)
    s2_ref[...] = jnp.concatenate([h2, a_s, a_d, z], axis=1)


def _tc2(acc1, w2p, b1p, tile8, a_src2, a_dst2, blk, grid):
    n = acc1.shape[1]
    return pl.pallas_call(
        _tc2_body,
        grid=(grid,),
        in_specs=[
            pl.BlockSpec((2, blk, 128), lambda i: (0, i, 0)),
            pl.BlockSpec((64, 1), lambda i: (0, 0)),
            pl.BlockSpec((1, 64), lambda i: (0, 0)),
            pl.BlockSpec((8, 64), lambda i: (0, 0)),
            pl.BlockSpec((1, 1), lambda i: (0, 0)),
            pl.BlockSpec((1, 1), lambda i: (0, 0)),
        ],
        out_specs=pl.BlockSpec((blk, 16), lambda i: (i, 0)),
        out_shape=jax.ShapeDtypeStruct((n, 16), jnp.float32),
    )(acc1, w2p, b1p, tile8, a_src2, a_dst2)


def _tc3_body(acc_ref, pnum_ref, pden_ref, b2_ref, out_ref):
    a = acc_ref[0] + acc_ref[1]
    num = jnp.dot(a, pnum_ref[...], preferred_element_type=jnp.float32)
    den = jnp.dot(a, pden_ref[...], preferred_element_type=jnp.float32)
    out_ref[...] = num / (den + 1e-16) + b2_ref[...]


def _tc3(acc2r, pnum, pden, b2):
    nr = acc2r.shape[1]
    blk8 = nr
    return pl.pallas_call(
        _tc3_body,
        grid=(1,),
        in_specs=[
            pl.BlockSpec((2, blk8, 128), lambda i: (0, 0, 0)),
            pl.BlockSpec((128, 8), lambda i: (0, 0)),
            pl.BlockSpec((128, 8), lambda i: (0, 0)),
            pl.BlockSpec((1, 1), lambda i: (0, 0)),
        ],
        out_specs=pl.BlockSpec((blk8, 8), lambda i: (0, 0)),
        out_shape=jax.ShapeDtypeStruct((nr, 8), jnp.float32),
    )(acc2r, pnum, pden, b2)


K1 = 128


def _make_sc1(n, e):
    n_chunks = e // K1
    rows_per_tile = n // NS
    npt = (n_chunks + NC * NS - 1) // (NC * NS)
    mesh = plsc.VectorSubcoreMesh(core_axis_name="c", subcore_axis_name="s")

    @functools.partial(
        pl.kernel,
        out_type=jax.ShapeDtypeStruct((NC, n, 128), jnp.float32),
        mesh=mesh,
        compiler_params=pltpu.CompilerParams(use_tc_tiling_on_sc=False),
        scratch_types=[
            pltpu.VMEM((npt, 128), jnp.int32),
            pltpu.VMEM((npt, 128), jnp.int32),
            pltpu.VMEM((2, K1, 80), jnp.float32),
            pltpu.VMEM((2, K1, 16), jnp.float32),
            pltpu.VMEM((2, K1, 80), jnp.float32),
            pltpu.VMEM((25, 80), jnp.float32),
            pltpu.VMEM_SHARED((n, 80), jnp.float32),
            pltpu.SemaphoreType.DMA((2,)),
            pltpu.SemaphoreType.DMA((2,)),
        ],
    )
    def sc1(s1_hbm, d1_hbm, src_hbm, dst_hbm, acc_hbm,
            src_i, dst_i, srow, drow, orow, zbuf, acc_sh, gsem, ssem):
        c = lax.axis_index("c")
        s = lax.axis_index("s")
        tile = c * NS + s
        base = tile * npt

        iota = lax.iota(jnp.int32, L)
        low8 = jnp.bitwise_and(iota, 7)
        zeros = jnp.zeros((L,), jnp.float32)

        def zz(i, _):
            for q in range(5):
                zbuf[i, pl.ds(q * 16, 16)] = zeros
            return 0
        lax.fori_loop(0, 25, zz, 0)

        idx_cps = [pltpu.async_copy(src_hbm.at[pl.ds(base, npt)], src_i,
                                    gsem.at[0]),
                   pltpu.async_copy(dst_hbm.at[pl.ds(base, npt)], dst_i,
                                    gsem.at[0])]
        zero_cps = [
            pltpu.async_copy(
                zbuf, acc_sh.at[pl.ds(s * rows_per_tile + q * 25, 25)],
                gsem.at[1])
            for q in range(rows_per_tile // 25)]
        for cp in idx_cps + zero_cps:
            cp.wait()
        plsc.subcore_barrier()

        def fire(j, b):
            @pl.when((j < npt) & (base + j < n_chunks))
            def _():
                pltpu.async_copy(s1_hbm.at[src_i.at[j]], srow.at[b],
                                 gsem.at[b])
                pltpu.async_copy(d1_hbm.at[dst_i.at[j]], drow.at[b],
                                 gsem.at[b])

        fire(0, 0)

        def chunk_loop(j, _):
            b = jnp.bitwise_and(j, 1)
            ok = base + j < n_chunks
            fire(j + 1, 1 - b)

            @pl.when((j >= 2) & (base + j - 2 < n_chunks))
            def _():
                pltpu.make_async_copy(orow.at[b],
                                      acc_sh.at[dst_i.at[j - 2]],
                                      ssem.at[b]).wait()

            @pl.when(ok)
            def _():
                pltpu.make_async_copy(s1_hbm.at[src_i.at[j]],
                                      srow.at[b], gsem.at[b]).wait()
                pltpu.make_async_copy(d1_hbm.at[dst_i.at[j]],
                                      drow.at[b], gsem.at[b]).wait()

                @plsc.parallel_loop(0, K1, 1, unroll=4)
                def edge(i):
                    v1 = srow[b, i, pl.ds(64, 16)]
                    v2 = drow[b, i, pl.ds(0, 16)]
                    al = v1 + v2
                    w = jnp.exp(jnp.maximum(al, 0.2 * al))
                    orow[b, i, pl.ds(0, 16)] = w
                    wd = w.at[low8].get(mode="promise_in_bounds")
                    for q in range(4):
                        orow[b, i, pl.ds(16 + q * 16, 16)] = (
                            srow[b, i, pl.ds(q * 16, 16)] * wd)

                pltpu.async_copy(orow.at[b], acc_sh.at[dst_i.at[j]],
                                 ssem.at[b], add=True)
            return 0
        lax.fori_loop(0, npt, chunk_loop, 0)

        for d in (2, 1):
            j = npt - d

            @pl.when(base + j < n_chunks)
            def _():
                pltpu.make_async_copy(orow.at[j % 2],
                                      acc_sh.at[dst_i.at[j]],
                                      ssem.at[j % 2]).wait()

        plsc.subcore_barrier()
        start = s * 624

        @pl.when(s < NS - 1)
        def _():
            pltpu.sync_copy(acc_sh.at[pl.ds(start, 624)],
                            acc_hbm.at[c, pl.ds(start, 624), pl.ds(0, 80)])

        @pl.when(s == NS - 1)
        def _():
            last = 624 * (NS - 1)
            pltpu.sync_copy(acc_sh.at[pl.ds(last, n - 624 * (NS - 1))],
                            acc_hbm.at[c, pl.ds(last, n - 624 * (NS - 1)),
                                       pl.ds(0, 80)])

    return sc1


K2 = 128


def _make_sc2(n, e):
    n_chunks = e // K2
    rows_per_tile = n // NS
    npt = (n_chunks + NC * NS - 1) // (NC * NS)
    mesh = plsc.VectorSubcoreMesh(core_axis_name="c", subcore_axis_name="s")

    @functools.partial(
        pl.kernel,
        out_type=jax.ShapeDtypeStruct((NC, n, 16), jnp.float32),
        mesh=mesh,
        compiler_params=pltpu.CompilerParams(use_tc_tiling_on_sc=False),
        scratch_types=[
            pltpu.VMEM((npt, 128), jnp.int32),
            pltpu.VMEM((npt, 128), jnp.int32),
            pltpu.VMEM((2, K2, 16), jnp.float32),
            pltpu.VMEM((2, K2, 16), jnp.float32),
            pltpu.VMEM((2, K2, 16), jnp.float32),
            pltpu.VMEM((25, 16), jnp.float32),
            pltpu.VMEM_SHARED((n, 16), jnp.float32),
            pltpu.SemaphoreType.DMA((2,)),
            pltpu.SemaphoreType.DMA((2,)),
        ],
    )
    def sc2(s2_hbm, src_hbm, dst_hbm, acc_hbm,
            src_i, dst_i, srow, drow, orow, zbuf, acc_sh, gsem, ssem):
        c = lax.axis_index("c")
        s = lax.axis_index("s")
        tile = c * NS + s
        base = tile * npt

        iota = lax.iota(jnp.int32, L)
        zeros = jnp.zeros((L,), jnp.float32)
        ones_i = jnp.full((L,), 1, jnp.int32)

        def zz(i, _):
            zbuf[i, pl.ds(0, 16)] = zeros
            return 0
        lax.fori_loop(0, 25, zz, 0)

        idx_cps = [pltpu.async_copy(src_hbm.at[pl.ds(base, npt)], src_i,
                                    gsem.at[0]),
                   pltpu.async_copy(dst_hbm.at[pl.ds(base, npt)], dst_i,
                                    gsem.at[0])]
        zero_cps = [
            pltpu.async_copy(
                zbuf, acc_sh.at[pl.ds(s * rows_per_tile + q * 25, 25)],
                gsem.at[1])
            for q in range(rows_per_tile // 25)]
        for cp in idx_cps + zero_cps:
            cp.wait()
        plsc.subcore_barrier()

        def fire(j, b):
            @pl.when((j < npt) & (base + j < n_chunks))
            def _():
                pltpu.async_copy(s2_hbm.at[src_i.at[j]], srow.at[b],
                                 gsem.at[b])
                pltpu.async_copy(s2_hbm.at[dst_i.at[j]], drow.at[b],
                                 gsem.at[b])

        fire(0, 0)

        def chunk_loop(j, _):
            b = jnp.bitwise_and(j, 1)
            ok = base + j < n_chunks
            fire(j + 1, 1 - b)

            @pl.when((j >= 2) & (base + j - 2 < n_chunks))
            def _():
                pltpu.make_async_copy(orow.at[b],
                                      acc_sh.at[dst_i.at[j - 2]],
                                      ssem.at[b]).wait()

            @pl.when(ok)
            def _():
                pltpu.make_async_copy(s2_hbm.at[src_i.at[j]],
                                      srow.at[b], gsem.at[b]).wait()
                pltpu.make_async_copy(s2_hbm.at[dst_i.at[j]],
                                      drow.at[b], gsem.at[b]).wait()

                @plsc.parallel_loop(0, K2, 1, unroll=8)
                def edge(i):
                    v_s = srow[b, i, pl.ds(0, 16)]
                    v_d = drow[b, i, pl.ds(0, 16)]
                    b_ad = v_d.at[ones_i + 1].get(mode="promise_in_bounds")
                    al = v_s + b_ad
                    wv = jnp.exp(jnp.maximum(al, 0.2 * al))
                    b_w = wv.at[ones_i].get(mode="promise_in_bounds")
                    prod = b_w * v_s
                    row = jnp.where(iota == 0, prod,
                                    jnp.where(iota == 1, b_w, zeros))
                    orow[b, i, pl.ds(0, 16)] = row

                pltpu.async_copy(orow.at[b], acc_sh.at[dst_i.at[j]],
                                 ssem.at[b], add=True)
            return 0
        lax.fori_loop(0, npt, chunk_loop, 0)

        for d in (2, 1):
            j = npt - d

            @pl.when(base + j < n_chunks)
            def _():
                pltpu.make_async_copy(orow.at[j % 2],
                                      acc_sh.at[dst_i.at[j]],
                                      ssem.at[j % 2]).wait()

        plsc.subcore_barrier()
        start = s * 624

        @pl.when(s < NS - 1)
        def _():
            pltpu.sync_copy(acc_sh.at[pl.ds(start, 624)],
                            acc_hbm.at[c, pl.ds(start, 624), :])

        @pl.when(s == NS - 1)
        def _():
            last = 624 * (NS - 1)
            pltpu.sync_copy(acc_sh.at[pl.ds(last, n - 624 * (NS - 1))],
                            acc_hbm.at[c, pl.ds(last, n - 624 * (NS - 1)), :])

    return sc2


def kernel(x, edge_index, W1, a_src1, a_dst1, b1, W2, a_src2, a_dst2, b2):
    n = x.shape[0]
    e = edge_index.shape[1]

    j = jnp.arange(64)
    perm = (j % 8) * 8 + j // 8
    w1p = W1[:, perm]
    w2p = W2[perm, :]
    b1p = b1[perm][None, :]
    eye8 = jnp.eye(8, dtype=jnp.float32)
    asm = (a_src1.T[:, :, None] * eye8[None, :, :]).reshape(64, 8)
    adm = (a_dst1.T[:, :, None] * eye8[None, :, :]).reshape(64, 8)
    tile8 = jnp.tile(eye8, (1, 8))

    n_chunks = e // 128
    npt = (n_chunks + NC * NS - 1) // (NC * NS)
    pad_rows = npt * NC * NS - n_chunks
    ei_p = lax.pad(edge_index, jnp.int32(0),
                   ((0, 0, 0), (0, pad_rows * 128, 0)))
    ei_p = ei_p.reshape(2, n_chunks + pad_rows, 128)
    src_r = ei_p[0]
    dst_r = ei_p[1]

    blk, grid = 1000, n // 1000

    s1, d1 = _tc1(x, w1p, asm, adm, blk, grid)
    acc1 = _make_sc1(n, e)(s1, d1, src_r, dst_r)
    s2 = _tc2(acc1, w2p, b1p, tile8, a_src2, a_dst2, blk, grid)
    acc2 = _make_sc2(n, e)(s2, src_r, dst_r)
    acc2r = acc2.reshape(2, n // 8, 128)
    eye128 = jnp.eye(128, dtype=jnp.float32)
    pnum = eye128[:, 0::16]
    pden = eye128[:, 1::16]
    out = _tc3(acc2r, pnum, pden, b2.reshape(1, 1))
    return out.reshape(n)


# speedup vs baseline: 274.0213x; 1.1697x over previous
import functools
import jax
import jax.numpy as jnp
from jax import lax
from jax.experimental import pallas as pl
from jax.experimental.pallas import tpu as pltpu
from jax.experimental.pallas import tpu_sc as plsc

NC = 2
NS = 16
L = 16


def _tc1_body(x_ref, w1p_ref, asm_ref, adm_ref, s1_ref, d1_ref):
    h = jnp.dot(x_ref[...], w1p_ref[...], preferred_element_type=jnp.float32)
    a_s = jnp.dot(h, asm_ref[...], preferred_element_type=jnp.float32)
    a_d = jnp.dot(h, adm_ref[...], preferred_element_type=jnp.float32)
    s1_ref[...] = jnp.concatenate([h, a_s, a_d], axis=1)
    d1_ref[...] = jnp.concatenate([a_d, jnp.zeros_like(a_d)], axis=1)


def _tc1(x, w1p, asm, adm, blk, grid):
    n = x.shape[0]
    f_in = x.shape[1]
    return pl.pallas_call(
        _tc1_body,
        grid=(grid,),
        in_specs=[
            pl.BlockSpec((blk, f_in), lambda i: (i, 0)),
            pl.BlockSpec((f_in, 64), lambda i: (0, 0)),
            pl.BlockSpec((64, 8), lambda i: (0, 0)),
            pl.BlockSpec((64, 8), lambda i: (0, 0)),
        ],
        out_specs=[
            pl.BlockSpec((blk, 80), lambda i: (i, 0)),
            pl.BlockSpec((blk, 16), lambda i: (i, 0)),
        ],
        out_shape=[
            jax.ShapeDtypeStruct((n, 80), jnp.float32),
            jax.ShapeDtypeStruct((n, 16), jnp.float32),
        ],
    )(x, w1p, asm, adm)


def _tc2_body(acc_ref, w2p_ref, b1p_ref, tile_ref, as2_ref, ad2_ref, s2_ref):
    a = acc_ref[0] + acc_ref[1]
    denom = a[:, 0:8]
    num = a[:, 16:80]
    dt = jnp.dot(denom, tile_ref[...], preferred_element_type=jnp.float32)
    out1 = jnp.maximum(num / (dt + 1e-16) + b1p_ref[...], 0.0)
    h2 = jnp.dot(out1, w2p_ref[...], preferred_element_type=jnp.float32)
    s2_ref[...] = jnp.concatenate(
        [h2, h2 * as2_ref[...], h2 * ad2_ref[...]], axis=1)


def _tc2(acc1, w2p, b1p, tile8, a_src2, a_dst2, blk, grid):
    n = acc1.shape[1]
    return pl.pallas_call(
        _tc2_body,
        grid=(grid,),
        in_specs=[
            pl.BlockSpec((2, blk, 128), lambda i: (0, i, 0)),
            pl.BlockSpec((64, 1), lambda i: (0, 0)),
            pl.BlockSpec((1, 64), lambda i: (0, 0)),
            pl.BlockSpec((8, 64), lambda i: (0, 0)),
            pl.BlockSpec((1, 1), lambda i: (0, 0)),
            pl.BlockSpec((1, 1), lambda i: (0, 0)),
        ],
        out_specs=pl.BlockSpec((blk, 3), lambda i: (i, 0)),
        out_shape=jax.ShapeDtypeStruct((n, 3), jnp.float32),
    )(acc1, w2p, b1p, tile8, a_src2, a_dst2)


def _tc3_body(acc_ref, b2_ref, out_ref):
    a = acc_ref[0] + acc_ref[1]
    out_ref[...] = a[0:1, :] / (a[1:2, :] + 1e-16) + b2_ref[...]


def _tc3(acc2, b2):
    n = acc2.shape[2]
    return pl.pallas_call(
        _tc3_body,
        grid=(1,),
        in_specs=[
            pl.BlockSpec((2, 2, n), lambda i: (0, 0, 0)),
            pl.BlockSpec((1, 1), lambda i: (0, 0)),
        ],
        out_specs=pl.BlockSpec((1, n), lambda i: (0, 0)),
        out_shape=jax.ShapeDtypeStruct((1, n), jnp.float32),
    )(acc2, b2)


K1 = 128


def _make_sc1(n, e):
    n_chunks = e // K1
    rows_per_tile = n // NS
    npt = (n_chunks + NC * NS - 1) // (NC * NS)
    mesh = plsc.VectorSubcoreMesh(core_axis_name="c", subcore_axis_name="s")

    @functools.partial(
        pl.kernel,
        out_type=jax.ShapeDtypeStruct((NC, n, 128), jnp.float32),
        mesh=mesh,
        compiler_params=pltpu.CompilerParams(use_tc_tiling_on_sc=False),
        scratch_types=[
            pltpu.VMEM((npt, 128), jnp.int32),
            pltpu.VMEM((npt, 128), jnp.int32),
            pltpu.VMEM((2, K1, 80), jnp.float32),
            pltpu.VMEM((2, K1, 16), jnp.float32),
            pltpu.VMEM((2, K1, 80), jnp.float32),
            pltpu.VMEM((25, 80), jnp.float32),
            pltpu.VMEM_SHARED((n, 80), jnp.float32),
            pltpu.SemaphoreType.DMA((2,)),
            pltpu.SemaphoreType.DMA((2,)),
        ],
    )
    def sc1(s1_hbm, d1_hbm, src_hbm, dst_hbm, acc_hbm,
            src_i, dst_i, srow, drow, orow, zbuf, acc_sh, gsem, ssem):
        c = lax.axis_index("c")
        s = lax.axis_index("s")
        tile = c * NS + s
        base = tile * npt

        iota = lax.iota(jnp.int32, L)
        low8 = jnp.bitwise_and(iota, 7)
        zeros = jnp.zeros((L,), jnp.float32)

        def zz(i, _):
            for q in range(5):
                zbuf[i, pl.ds(q * 16, 16)] = zeros
            return 0
        lax.fori_loop(0, 25, zz, 0)

        idx_cps = [pltpu.async_copy(src_hbm.at[pl.ds(base, npt)], src_i,
                                    gsem.at[0]),
                   pltpu.async_copy(dst_hbm.at[pl.ds(base, npt)], dst_i,
                                    gsem.at[0])]
        zero_cps = [
            pltpu.async_copy(
                zbuf, acc_sh.at[pl.ds(s * rows_per_tile + q * 25, 25)],
                gsem.at[1])
            for q in range(rows_per_tile // 25)]
        for cp in idx_cps + zero_cps:
            cp.wait()
        plsc.subcore_barrier()

        def fire(j, b):
            @pl.when((j < npt) & (base + j < n_chunks))
            def _():
                pltpu.async_copy(s1_hbm.at[src_i.at[j]], srow.at[b],
                                 gsem.at[b])
                pltpu.async_copy(d1_hbm.at[dst_i.at[j]], drow.at[b],
                                 gsem.at[b])

        fire(0, 0)

        def chunk_loop(j, _):
            b = jnp.bitwise_and(j, 1)
            ok = base + j < n_chunks
            fire(j + 1, 1 - b)

            @pl.when((j >= 2) & (base + j - 2 < n_chunks))
            def _():
                pltpu.make_async_copy(orow.at[b],
                                      acc_sh.at[dst_i.at[j - 2]],
                                      ssem.at[b]).wait()

            @pl.when(ok)
            def _():
                pltpu.make_async_copy(s1_hbm.at[src_i.at[j]],
                                      srow.at[b], gsem.at[b]).wait()
                pltpu.make_async_copy(d1_hbm.at[dst_i.at[j]],
                                      drow.at[b], gsem.at[b]).wait()

                @plsc.parallel_loop(0, K1, 1, unroll=4)
                def edge(i):
                    v1 = srow[b, i, pl.ds(64, 16)]
                    v2 = drow[b, i, pl.ds(0, 16)]
                    al = v1 + v2
                    w = jnp.exp(jnp.maximum(al, 0.2 * al))
                    orow[b, i, pl.ds(0, 16)] = w
                    wd = w.at[low8].get(mode="promise_in_bounds")
                    for q in range(4):
                        orow[b, i, pl.ds(16 + q * 16, 16)] = (
                            srow[b, i, pl.ds(q * 16, 16)] * wd)

                pltpu.async_copy(orow.at[b], acc_sh.at[dst_i.at[j]],
                                 ssem.at[b], add=True)
            return 0
        lax.fori_loop(0, npt, chunk_loop, 0)

        for d in (2, 1):
            j = npt - d

            @pl.when(base + j < n_chunks)
            def _():
                pltpu.make_async_copy(orow.at[j % 2],
                                      acc_sh.at[dst_i.at[j]],
                                      ssem.at[j % 2]).wait()

        plsc.subcore_barrier()
        start = s * 624

        @pl.when(s < NS - 1)
        def _():
            pltpu.sync_copy(acc_sh.at[pl.ds(start, 624)],
                            acc_hbm.at[c, pl.ds(start, 624), pl.ds(0, 80)])

        @pl.when(s == NS - 1)
        def _():
            last = 624 * (NS - 1)
            pltpu.sync_copy(acc_sh.at[pl.ds(last, n - 624 * (NS - 1))],
                            acc_hbm.at[c, pl.ds(last, n - 624 * (NS - 1)),
                                       pl.ds(0, 80)])

    return sc1


K2 = 128


def _make_sc2(n, e):
    n_chunks = e // K2
    npt = (n_chunks + NC * NS - 1) // (NC * NS)
    np128 = ((n + 127) // 128) * 128
    mesh = plsc.VectorSubcoreMesh(core_axis_name="c", subcore_axis_name="s")

    @functools.partial(
        pl.kernel,
        out_type=jax.ShapeDtypeStruct((NC, 2, n), jnp.float32),
        mesh=mesh,
        compiler_params=pltpu.CompilerParams(use_tc_tiling_on_sc=False,
                                             needs_layout_passes=False),
        scratch_types=[
            pltpu.VMEM((npt, 128), jnp.int32),
            pltpu.VMEM((npt, 128), jnp.int32),
            pltpu.VMEM((np128,), jnp.float32),
            pltpu.VMEM((np128,), jnp.float32),
            pltpu.VMEM((np128,), jnp.float32),
            pltpu.VMEM((np128,), jnp.float32),
            pltpu.VMEM((np128,), jnp.float32),
            pltpu.VMEM((NS, 2, 640), jnp.float32),
            pltpu.VMEM_SHARED((NS, 2, n), jnp.float32),
            pltpu.SemaphoreType.DMA((2,)),
        ],
    )
    def sc2(s2t_hbm, src_hbm, dst_hbm, acc_hbm,
            src_i, dst_i, h2v, asv, adv, nacc, dacc, rbuf, pacc_sh, gsem):
        c = lax.axis_index("c")
        s = lax.axis_index("s")
        tile = c * NS + s
        base = tile * npt

        zeros = jnp.zeros((L,), jnp.float32)

        cps = [pltpu.async_copy(src_hbm.at[pl.ds(base, npt)], src_i,
                                gsem.at[0]),
               pltpu.async_copy(dst_hbm.at[pl.ds(base, npt)], dst_i,
                                gsem.at[0]),
               pltpu.async_copy(s2t_hbm.at[0], h2v.at[pl.ds(0, n)],
                                gsem.at[1]),
               pltpu.async_copy(s2t_hbm.at[1], asv.at[pl.ds(0, n)],
                                gsem.at[1]),
               pltpu.async_copy(s2t_hbm.at[2], adv.at[pl.ds(0, n)],
                                gsem.at[1])]

        def zz(i, _):
            nacc[pl.ds(i * 16, 16)] = zeros
            dacc[pl.ds(i * 16, 16)] = zeros
            return 0
        lax.fori_loop(0, np128 // 16, zz, 0)
        for cp in cps:
            cp.wait()

        def chunk_loop(j, _):
            @pl.when(base + j < n_chunks)
            def _():
                for g in range(8):
                    src16 = src_i[j, pl.ds(g * 16, 16)]
                    dst16 = dst_i[j, pl.ds(g * 16, 16)]
                    a_s = plsc.load_gather(asv, [src16])
                    a_d = plsc.load_gather(adv, [dst16])
                    h16 = plsc.load_gather(h2v, [src16])
                    al = a_s + a_d
                    w = jnp.exp(jnp.maximum(al, 0.2 * al))
                    plsc.addupdate_scatter(nacc, [dst16], w * h16)
                    plsc.addupdate_scatter(dacc, [dst16], w)
            return 0
        lax.fori_loop(0, npt, chunk_loop, 0)

        pltpu.sync_copy(nacc.at[pl.ds(0, n)], pacc_sh.at[s, 0])
        pltpu.sync_copy(dacc.at[pl.ds(0, n)], pacc_sh.at[s, 1])
        plsc.subcore_barrier()

        start = s * 624
        cps = [pltpu.async_copy(pacc_sh.at[t2, c2, pl.ds(start, 640)],
                                rbuf.at[t2, c2], gsem.at[0])
               for t2 in range(NS) for c2 in range(2)]
        for cp in cps:
            cp.wait()

        for c2, buf in ((0, nacc), (1, dacc)):
            def red(v, _):
                acc16 = rbuf[0, c2, pl.ds(v * 16, 16)]
                for t2 in range(1, NS):
                    acc16 = acc16 + rbuf[t2, c2, pl.ds(v * 16, 16)]
                buf[pl.ds(v * 16, 16)] = acc16
                return 0
            lax.fori_loop(0, 40, red, 0)

        @pl.when(s < NS - 1)
        def _():
            pltpu.sync_copy(nacc.at[pl.ds(0, 624)],
                            acc_hbm.at[c, 0, pl.ds(start, 624)])
            pltpu.sync_copy(dacc.at[pl.ds(0, 624)],
                            acc_hbm.at[c, 1, pl.ds(start, 624)])

        @pl.when(s == NS - 1)
        def _():
            last = 624 * (NS - 1)
            pltpu.sync_copy(nacc.at[pl.ds(0, n - last)],
                            acc_hbm.at[c, 0, pl.ds(last, n - last)])
            pltpu.sync_copy(dacc.at[pl.ds(0, n - last)],
                            acc_hbm.at[c, 1, pl.ds(last, n - last)])

    return sc2


def kernel(x, edge_index, W1, a_src1, a_dst1, b1, W2, a_src2, a_dst2, b2):
    n = x.shape[0]
    e = edge_index.shape[1]

    j = jnp.arange(64)
    perm = (j % 8) * 8 + j // 8
    w1p = W1[:, perm]
    w2p = W2[perm, :]
    b1p = b1[perm][None, :]
    eye8 = jnp.eye(8, dtype=jnp.float32)
    asm = (a_src1.T[:, :, None] * eye8[None, :, :]).reshape(64, 8)
    adm = (a_dst1.T[:, :, None] * eye8[None, :, :]).reshape(64, 8)
    tile8 = jnp.tile(eye8, (1, 8))

    n_chunks = e // 128
    npt = (n_chunks + NC * NS - 1) // (NC * NS)
    pad_rows = npt * NC * NS - n_chunks
    ei_p = lax.pad(edge_index, jnp.int32(0),
                   ((0, 0, 0), (0, pad_rows * 128, 0)))
    ei_p = ei_p.reshape(2, n_chunks + pad_rows, 128)
    src_r = ei_p[0]
    dst_r = ei_p[1]

    blk, grid = 1000, n // 1000

    s1, d1 = _tc1(x, w1p, asm, adm, blk, grid)
    acc1 = _make_sc1(n, e)(s1, d1, src_r, dst_r)
    s2 = _tc2(acc1, w2p, b1p, tile8, a_src2, a_dst2, blk, grid)
    acc2 = _make_sc2(n, e)(s2.T, src_r, dst_r)
    out = _tc3(acc2, b2.reshape(1, 1))
    return out.reshape(n)
